# Initial kernel scaffold; baseline (speedup 1.0000x reference)
import jax
import jax.numpy as jnp
from jax.experimental import pallas as pl


def kernel(x0, x1, edge_index, enc0_W1, enc0_b1, enc0_g1, enc0_bb1, enc0_W2, enc0_b2, enc0_rg, enc0_rb, enc1_W1, enc1_b1, enc1_g1, enc1_bb1, enc1_W2, enc1_b2, enc1_rg, enc1_rb, comb_W, comb_b, comb_g, comb_bb, dec0_W, dec0_b, dec1_W, dec1_b, clf_W1, clf_b1, clf_W2, clf_b2):
    raise NotImplementedError("write your pallas kernel here")


# speedup vs baseline: 27.5459x; 27.5459x over previous
import functools

import jax
import jax.numpy as jnp
from jax import lax
from jax.experimental import pallas as pl
from jax.experimental.pallas import tpu as pltpu
from jax.experimental.pallas import tpu_sc as plsc

_N = 100000
_E = 1600000
_D = 128
_H = 64
_O = 16
_NC = 2
_NS = 16
_ER = _E // 128
_ROWBLK = 8
_NCH = _ER // _ROWBLK
_TAIL_ROW = _NCH * _ROWBLK
_TAIL_N = _ER - _TAIL_ROW
_C0_CH = _NCH // 2

_mesh = plsc.VectorSubcoreMesh(core_axis_name="c", subcore_axis_name="s",
                               num_cores=_NC, num_subcores=_NS)
_sc_params = pltpu.CompilerParams(use_tc_tiling_on_sc=False)

_OCT_Q, _OCT_R = divmod(_N // 8, _NS)


def _node_copy(sid, src_at, dst_at):
    base = pl.multiple_of((sid * _OCT_Q + jnp.minimum(sid, _OCT_R)) * 8, 8)
    pltpu.sync_copy(src_at(base, _OCT_Q * 8), dst_at(base, _OCT_Q * 8))

    @pl.when(sid < _OCT_R)
    def _():
        b1 = pl.multiple_of(base + _OCT_Q * 8, 8)
        pltpu.sync_copy(src_at(b1, 8), dst_at(b1, 8))


def _chunk_sched(nchunks, rm, sid):
    q = nchunks // _NS
    b = sid * q + jnp.minimum(sid, rm)
    n = q + jnp.where(sid < rm, 1, 0)
    return b, n


@functools.partial(
    pl.kernel,
    out_type=jax.ShapeDtypeStruct((_NC, _N, 1), jnp.float32),
    mesh=_mesh,
    compiler_params=_sc_params,
    scratch_types=[
        pltpu.VMEM((_ROWBLK, 128), jnp.int32),
        pltpu.VMEM((128, 1), jnp.float32),
        pltpu.VMEM_SHARED((_N, 1), jnp.float32),
    ],
)
def _deg_kernel(dst2_hbm, zcol_hbm, ones_hbm, out_hbm, didx, ones_v, acc):
    cid = lax.axis_index("c")
    sid = lax.axis_index("s")
    pltpu.sync_copy(ones_hbm, ones_v)
    _node_copy(sid, lambda b, n: zcol_hbm.at[pl.ds(b, n)],
               lambda b, n: acc.at[pl.ds(b, n)])
    plsc.subcore_barrier()

    chunk0 = cid * _C0_CH
    b, n = _chunk_sched(_C0_CH, _C0_CH % _NS, sid)

    def chunk_body(c, carry):
        row = pl.multiple_of((chunk0 + b + c) * _ROWBLK, _ROWBLK)
        pltpu.sync_copy(dst2_hbm.at[pl.ds(row, _ROWBLK)], didx)
        for j in range(_ROWBLK):
            pltpu.sync_copy(ones_v, acc.at[didx.at[j]], add=True)
        return carry

    lax.fori_loop(0, n, chunk_body, 0)

    @pl.when((cid == 1) & (sid == _NS - 1))
    def _():
        pltpu.sync_copy(dst2_hbm.at[pl.ds(_TAIL_ROW, _TAIL_N)],
                        didx.at[pl.ds(0, _TAIL_N)])
        for j in range(_TAIL_N):
            pltpu.sync_copy(ones_v, acc.at[didx.at[j]], add=True)

    plsc.subcore_barrier()
    _node_copy(sid, lambda b2, n2: acc.at[pl.ds(b2, n2)],
               lambda b2, n2: out_hbm.at[cid].at[pl.ds(b2, n2)])


def _make_agg_kernel(per_core_features):
    tab_shape = ((_NC, _N, _O),) if per_core_features else ((_N, _O),)

    @functools.partial(
        pl.kernel,
        out_type=jax.ShapeDtypeStruct((_NC, _N, _O), jnp.float32),
        mesh=_mesh,
        compiler_params=_sc_params,
        scratch_types=[
            pltpu.VMEM((_ROWBLK * 128,), jnp.int32),
            pltpu.VMEM((_ROWBLK, 128), jnp.int32),
            pltpu.VMEM((_ROWBLK * 128, _O), jnp.float32),
            pltpu.VMEM_SHARED((_N, _O), jnp.float32),
            pltpu.SemaphoreType.DMA,
        ],
    )
    def _agg(tab_hbm, src_hbm, dst2_hbm, out_hbm, sidx, didx, rows_v, acc, sem):
        cid = lax.axis_index("c")
        sid = lax.axis_index("s")
        tab = tab_hbm.at[cid] if per_core_features else tab_hbm
        _node_copy(sid, lambda b, n: tab.at[pl.ds(b, n)],
                   lambda b, n: acc.at[pl.ds(b, n)])
        plsc.subcore_barrier()

        if per_core_features:
            chunk0 = 0
            b, n = _chunk_sched(_NCH, _NCH % _NS, sid)
            tail_here = sid == _NS - 1
        else:
            chunk0 = cid * _C0_CH
            b, n = _chunk_sched(_C0_CH, _C0_CH % _NS, sid)
            tail_here = (cid == 1) & (sid == _NS - 1)

        def process(row, nrows):
            pltpu.sync_copy(src_hbm.at[pl.ds(row * 128, nrows * 128)],
                            sidx.at[pl.ds(0, nrows * 128)])
            pltpu.sync_copy(dst2_hbm.at[pl.ds(row, nrows)],
                            didx.at[pl.ds(0, nrows)])
            pltpu.async_copy(tab.at[sidx.at[pl.ds(0, nrows * 128)]],
                             rows_v.at[pl.ds(0, nrows * 128)], sem).wait()
            for j in range(nrows):
                pltpu.sync_copy(rows_v.at[pl.ds(j * 128, 128)],
                                acc.at[didx.at[j]], add=True)

        def chunk_body(c, carry):
            process(pl.multiple_of((chunk0 + b + c) * _ROWBLK, _ROWBLK),
                    _ROWBLK)
            return carry

        lax.fori_loop(0, n, chunk_body, 0)

        @pl.when(tail_here)
        def _():
            process(_TAIL_ROW, _TAIL_N)

        plsc.subcore_barrier()
        _node_copy(sid, lambda b2, n2: acc.at[pl.ds(b2, n2)],
                   lambda b2, n2: out_hbm.at[cid].at[pl.ds(b2, n2)])

    return _agg


_agg1_kernel = _make_agg_kernel(True)
_agg2_kernel = _make_agg_kernel(False)


def _gelu(x):
    return 0.5 * x * (1.0 + lax.erf(x * 0.7071067811865476))


def _ln(x, g, b, eps=1e-5):
    m = jnp.mean(x, axis=-1, keepdims=True)
    v = jnp.mean((x - m) ** 2, axis=-1, keepdims=True)
    return (x - m) / jnp.sqrt(v + eps) * g + b


_RB = 2000


def _full(shape):
    return pl.BlockSpec(shape, lambda i: tuple(0 for _ in shape))


def _enc_body(x0, x1, deg2, W10, b10, g10, bb10, W20, b20, rg0, rb0,
              W11, b11, g11, bb11, W21, b21, rg1, rb1, zcp, dinv):
    d = deg2[0] + deg2[1] + 1.0
    di = lax.rsqrt(d)

    def enc(x, W1, b1, g1, bb1, W2, b2, rg, rb):
        h = jnp.dot(x, W1, preferred_element_type=jnp.float32) + b1
        h = _ln(h, g1, bb1)
        h = _gelu(h)
        h2 = jnp.dot(h, W2, preferred_element_type=jnp.float32) + b2
        t = h2 + h2
        return _ln(t, rg, rb)

    zcp[0] = di * enc(x0[...], W10[...], b10[...], g10[...], bb10[...],
                      W20[...], b20[...], rg0[...], rb0[...])
    zcp[1] = di * enc(x1[...], W11[...], b11[...], g11[...], bb11[...],
                      W21[...], b21[...], rg1[...], rb1[...])
    dinv[...] = di


def _enc_call(x0, x1, deg2, w):
    grid = _N // _RB
    in_specs = [
        pl.BlockSpec((_RB, _D), lambda i: (i, 0)),
        pl.BlockSpec((_RB, _D), lambda i: (i, 0)),
        pl.BlockSpec((_NC, _RB, 1), lambda i: (0, i, 0)),
    ] + [_full(a.shape) for a in w]
    return pl.pallas_call(
        _enc_body,
        grid=(grid,),
        in_specs=in_specs,
        out_specs=[pl.BlockSpec((_NC, _RB, _O), lambda i: (0, i, 0)),
                   pl.BlockSpec((_RB, 1), lambda i: (i, 0))],
        out_shape=[jax.ShapeDtypeStruct((_NC, _N, _O), jnp.float32),
                   jax.ShapeDtypeStruct((_N, 1), jnp.float32)],
    )(x0, x1, deg2, *w)


def _mid_body(S, dinv, comb_W, comb_b, comb_g, comb_bb, clf_W1, clf_b1,
              clf_W2, clf_b2, z_out, zp_out, dom_out):
    di = dinv[...]
    agg = di * jnp.concatenate([S[0], S[1]], axis=1)
    t = jnp.dot(agg, comb_W[...], preferred_element_type=jnp.float32) + comb_b[...]
    z = _ln(t, comb_g[...], comb_bb[...])
    h = jnp.dot(z, clf_W1[...], preferred_element_type=jnp.float32) + clf_b1[...]
    h = _gelu(h)
    dom = jnp.dot(h, clf_W2[...], preferred_element_type=jnp.float32) + clf_b2[...]
    z_out[...] = z
    zp_out[...] = di * z
    dom_out[...] = dom


def _mid_call(S, dinv, w):
    grid = _N // _RB
    in_specs = [
        pl.BlockSpec((_NC, _RB, _O), lambda i: (0, i, 0)),
        pl.BlockSpec((_RB, 1), lambda i: (i, 0)),
    ] + [_full(a.shape) for a in w]
    return pl.pallas_call(
        _mid_body,
        grid=(grid,),
        in_specs=in_specs,
        out_specs=[pl.BlockSpec((_RB, _O), lambda i: (i, 0)),
                   pl.BlockSpec((_RB, _O), lambda i: (i, 0)),
                   pl.BlockSpec((_RB, 8), lambda i: (i, 0))],
        out_shape=[jax.ShapeDtypeStruct((_N, _O), jnp.float32),
                   jax.ShapeDtypeStruct((_N, _O), jnp.float32),
                   jax.ShapeDtypeStruct((_N, 8), jnp.float32)],
    )(S, dinv, *w)


def _fin_body(S2, zp, dinv, W0, b0, W1, b1, rec0, rec1):
    aggz = dinv[...] * (S2[0] + S2[1] - zp[...])
    rec0[...] = jnp.dot(aggz, W0[...], preferred_element_type=jnp.float32) + b0[...]
    rec1[...] = jnp.dot(aggz, W1[...], preferred_element_type=jnp.float32) + b1[...]


def _fin_call(S2, zp, dinv, w):
    grid = _N // _RB
    in_specs = [
        pl.BlockSpec((_NC, _RB, _O), lambda i: (0, i, 0)),
        pl.BlockSpec((_RB, _O), lambda i: (i, 0)),
        pl.BlockSpec((_RB, 1), lambda i: (i, 0)),
    ] + [_full(a.shape) for a in w]
    return pl.pallas_call(
        _fin_body,
        grid=(grid,),
        in_specs=in_specs,
        out_specs=[pl.BlockSpec((_RB, _D), lambda i: (i, 0)),
                   pl.BlockSpec((_RB, _D), lambda i: (i, 0))],
        out_shape=[jax.ShapeDtypeStruct((_N, _D), jnp.float32),
                   jax.ShapeDtypeStruct((_N, _D), jnp.float32)],
    )(S2, zp, dinv, *w)


def kernel(x0, x1, edge_index, enc0_W1, enc0_b1, enc0_g1, enc0_bb1, enc0_W2,
           enc0_b2, enc0_rg, enc0_rb, enc1_W1, enc1_b1, enc1_g1, enc1_bb1,
           enc1_W2, enc1_b2, enc1_rg, enc1_rb, comb_W, comb_b, comb_g,
           comb_bb, dec0_W, dec0_b, dec1_W, dec1_b, clf_W1, clf_b1, clf_W2,
           clf_b2):
    f32 = jnp.float32
    src = edge_index[0]
    dst2 = edge_index[1].reshape(_ER, 128)
    zcol = jnp.zeros((_N, 1), f32)
    ones128 = jnp.ones((128, 1), f32)

    deg2 = _deg_kernel(dst2, zcol, ones128)

    row = lambda a: a.reshape(1, -1)
    enc_w = (enc0_W1, row(enc0_b1), row(enc0_g1), row(enc0_bb1), enc0_W2,
             row(enc0_b2), row(enc0_rg), row(enc0_rb),
             enc1_W1, row(enc1_b1), row(enc1_g1), row(enc1_bb1), enc1_W2,
             row(enc1_b2), row(enc1_rg), row(enc1_rb))
    zcp, dinv = _enc_call(x0, x1, deg2, enc_w)

    S = _agg1_kernel(zcp, src, dst2)

    mid_w = (comb_W, row(comb_b), row(comb_g), row(comb_bb),
             clf_W1, row(clf_b1), clf_W2, row(clf_b2))
    z, zp, dom = _mid_call(S, dinv, mid_w)

    S2 = _agg2_kernel(zp, src, dst2)

    rec0, rec1 = _fin_call(S2, zp, dinv, (dec0_W, row(dec0_b),
                                          dec1_W, row(dec1_b)))
    return (z, rec0, rec1, dom)


# speedup vs baseline: 28.7461x; 1.0436x over previous
import functools

import jax
import jax.numpy as jnp
from jax import lax
from jax.experimental import pallas as pl
from jax.experimental.pallas import tpu as pltpu
from jax.experimental.pallas import tpu_sc as plsc

_N = 100000
_E = 1600000
_D = 128
_H = 64
_O = 16
_NC = 2
_NS = 16
_P = _N // 8
_ER = _E // 128
_RB = 4
_NCH = _ER // _RB
_C0_CH = 1562

_mesh = plsc.VectorSubcoreMesh(core_axis_name="c", subcore_axis_name="s",
                               num_cores=_NC, num_subcores=_NS)
_sc_params = pltpu.CompilerParams(use_tc_tiling_on_sc=False)

_OCT_Q, _OCT_R = divmod(_N // 8, _NS)


def _node_copy(sid, src_at, dst_at):
    base = pl.multiple_of((sid * _OCT_Q + jnp.minimum(sid, _OCT_R)) * 8, 8)
    pltpu.sync_copy(src_at(base, _OCT_Q * 8), dst_at(base, _OCT_Q * 8))

    @pl.when(sid < _OCT_R)
    def _():
        b1 = pl.multiple_of(base + _OCT_Q * 8, 8)
        pltpu.sync_copy(src_at(b1, 8), dst_at(b1, 8))


def _chunk_sched(nchunks, rm, sid):
    q = nchunks // _NS
    b = sid * q + jnp.minimum(sid, rm)
    n = q + jnp.where(sid < rm, 1, 0)
    return b, n


@functools.partial(
    pl.kernel,
    out_type=[jax.ShapeDtypeStruct((_N,), jnp.float32),
              jax.ShapeDtypeStruct((_N,), jnp.float32)],
    mesh=_mesh,
    compiler_params=_sc_params,
    scratch_types=[
        pltpu.VMEM((_RB, 128), jnp.int32),
        pltpu.VMEM((_RB, 128), jnp.int32),
        pltpu.VMEM((128,), jnp.float32),
        pltpu.VMEM_SHARED((_N,), jnp.float32),
        pltpu.SemaphoreType.DMA,
        pltpu.SemaphoreType.DMA,
    ],
)
def _deg_kernel(ei3_hbm, zcol_hbm, ones_hbm, out0_hbm, out1_hbm,
                didx0, didx1, ones_v, acc, sem0, sem1):
    cid = lax.axis_index("c")
    sid = lax.axis_index("s")
    pltpu.sync_copy(ones_hbm, ones_v)
    _node_copy(sid, lambda b, n: zcol_hbm.at[pl.ds(b, n)],
               lambda b, n: acc.at[pl.ds(b, n)])
    plsc.subcore_barrier()

    chunk0 = cid * _C0_CH
    b, n = _chunk_sched(_C0_CH, jnp.where(cid == 0, _C0_CH % _NS,
                                          (_NCH - _C0_CH) % _NS), sid)
    bufs = ((didx0, sem0), (didx1, sem1))

    def load(c, k):
        row = pl.multiple_of((chunk0 + b + c) * _RB, _RB)
        pltpu.sync_copy(ei3_hbm.at[1].at[pl.ds(row, _RB)], bufs[k][0])

    def fire(k):
        didx, sem = bufs[k]
        return [pltpu.async_copy(ones_v, acc.at[didx.at[j]], sem, add=True)
                for j in range(_RB)]

    def drain(k):
        didx, sem = bufs[k]
        for j in range(_RB):
            pltpu.make_async_copy(ones_v, acc.at[didx.at[j]], sem).wait()

    load(0, 0)
    fire(0)

    def pair(p, carry):
        c0 = 2 * p
        load(c0 + 1, 1)
        drain(0)
        fire(1)

        @pl.when(c0 + 2 < n)
        def _():
            load(c0 + 2, 0)
            fire(0)

        drain(1)
        return carry

    lax.fori_loop(0, n // 2, pair, 0)

    @pl.when(n % 2 == 1)
    def _():
        drain(0)

    plsc.subcore_barrier()

    @pl.when(cid == 0)
    def _():
        _node_copy(sid, lambda b2, n2: acc.at[pl.ds(b2, n2)],
                   lambda b2, n2: out0_hbm.at[pl.ds(b2, n2)])

    @pl.when(cid == 1)
    def _():
        _node_copy(sid, lambda b2, n2: acc.at[pl.ds(b2, n2)],
                   lambda b2, n2: out1_hbm.at[pl.ds(b2, n2)])


def _make_agg_kernel(per_core_features):
    @functools.partial(
        pl.kernel,
        out_type=jax.ShapeDtypeStruct((_NC, _N, _O), jnp.float32),
        mesh=_mesh,
        compiler_params=_sc_params,
        scratch_types=[
            pltpu.VMEM((_RB, 128), jnp.int32),
            pltpu.VMEM((_RB, 128), jnp.int32),
            pltpu.VMEM((_RB, 128), jnp.int32),
            pltpu.VMEM((_RB, 128), jnp.int32),
            pltpu.VMEM((_RB * 128, _O), jnp.float32),
            pltpu.VMEM((_RB * 128, _O), jnp.float32),
            pltpu.VMEM_SHARED((_N, _O), jnp.float32),
            pltpu.SemaphoreType.DMA,
            pltpu.SemaphoreType.DMA,
            pltpu.SemaphoreType.DMA,
            pltpu.SemaphoreType.DMA,
        ],
    )
    def _agg(tab_hbm, ei3_hbm, out_hbm, sidx0, didx0, sidx1, didx1,
             rows0, rows1, acc, gs0, ss0, gs1, ss1):
        cid = lax.axis_index("c")
        sid = lax.axis_index("s")
        tab = tab_hbm.at[cid] if per_core_features else tab_hbm
        _node_copy(sid, lambda b, n: tab.at[pl.ds(b, n)],
                   lambda b, n: acc.at[pl.ds(b, n)])
        plsc.subcore_barrier()

        if per_core_features:
            chunk0 = 0
            b, n = _chunk_sched(_NCH, _NCH % _NS, sid)
        else:
            chunk0 = cid * _C0_CH
            b, n = _chunk_sched(_C0_CH, jnp.where(cid == 0, _C0_CH % _NS,
                                                  (_NCH - _C0_CH) % _NS), sid)

        bufs = ((sidx0, didx0, rows0, gs0, ss0),
                (sidx1, didx1, rows1, gs1, ss1))

        def load_fire(c, k):
            sidx, didx, rows, gsem, _ = bufs[k]
            row = pl.multiple_of((chunk0 + b + c) * _RB, _RB)
            pltpu.sync_copy(ei3_hbm.at[0].at[pl.ds(row, _RB)], sidx)
            pltpu.sync_copy(ei3_hbm.at[1].at[pl.ds(row, _RB)], didx)
            for j in range(_RB):
                pltpu.async_copy(tab.at[sidx.at[j]],
                                 rows.at[pl.ds(j * 128, 128)], gsem)

        def scatter(k):
            sidx, didx, rows, gsem, ssem = bufs[k]
            for j in range(_RB):
                pltpu.make_async_copy(tab.at[sidx.at[j]],
                                      rows.at[pl.ds(j * 128, 128)],
                                      gsem).wait()
            for j in range(_RB):
                pltpu.async_copy(rows.at[pl.ds(j * 128, 128)],
                                 acc.at[didx.at[j]], ssem, add=True)

        def drain(k):
            sidx, didx, rows, _, ssem = bufs[k]
            for j in range(_RB):
                pltpu.make_async_copy(rows.at[pl.ds(j * 128, 128)],
                                      acc.at[didx.at[j]], ssem).wait()

        load_fire(0, 0)

        def pair(p, carry):
            c0 = 2 * p
            load_fire(c0 + 1, 1)
            scatter(0)
            drain(0)

            @pl.when(c0 + 2 < n)
            def _():
                load_fire(c0 + 2, 0)

            scatter(1)
            drain(1)
            return carry

        lax.fori_loop(0, n // 2, pair, 0)

        @pl.when(n % 2 == 1)
        def _():
            scatter(0)
            drain(0)

        plsc.subcore_barrier()
        _node_copy(sid, lambda b2, n2: acc.at[pl.ds(b2, n2)],
                   lambda b2, n2: out_hbm.at[cid].at[pl.ds(b2, n2)])

    return _agg


_agg1_kernel = _make_agg_kernel(True)
_agg2_kernel = _make_agg_kernel(False)


def _gelu(x):
    return 0.5 * x * (1.0 + lax.erf(x * 0.7071067811865476))


def _ln(x, g, b, eps=1e-5):
    m = jnp.mean(x, axis=-1, keepdims=True)
    v = jnp.mean((x - m) ** 2, axis=-1, keepdims=True)
    return (x - m) / jnp.sqrt(v + eps) * g + b


_RN = 1984
_RP = _RN // 8
_GRID = (_N + _RN - 1) // _RN


def _pack16(z, scale8=None):
    z3 = z.reshape(_RP, 8, _O)
    parts = []
    for s in range(8):
        p = z3[:, s, :]
        if scale8 is not None:
            p = p * scale8[:, s:s + 1]
        parts.append(p)
    return jnp.concatenate(parts, axis=1)


def _unpack16(zp):
    parts = [zp[:, 16 * s:16 * (s + 1)].reshape(_RP, 1, _O) for s in range(8)]
    return jnp.concatenate(parts, axis=1).reshape(_RN, _O)


def _unpack_col(d8):
    parts = [d8[:, s:s + 1].reshape(_RP, 1, 1) for s in range(8)]
    return jnp.concatenate(parts, axis=1).reshape(_RN, 1)


def _full(shape):
    return pl.BlockSpec(shape, lambda i: tuple(0 for _ in shape))


def _enc_body(x0, x1, d0, d1, W10, b10, g10, bb10, W20, b20, rg0, rb0,
              W11, b11, g11, bb11, W21, b21, rg1, rb1, zcp, dinv8):
    di8 = lax.rsqrt(d0[...] + d1[...] + 1.0)

    def enc(x, W1, b1, g1, bb1, W2, b2, rg, rb):
        h = jnp.dot(x, W1, preferred_element_type=jnp.float32) + b1
        h = _ln(h, g1, bb1)
        h = _gelu(h)
        h2 = jnp.dot(h, W2, preferred_element_type=jnp.float32) + b2
        t = h2 + h2
        return _ln(t, rg, rb)

    z0 = enc(x0[...], W10[...], b10[...], g10[...], bb10[...],
             W20[...], b20[...], rg0[...], rb0[...])
    z1 = enc(x1[...], W11[...], b11[...], g11[...], bb11[...],
             W21[...], b21[...], rg1[...], rb1[...])
    zcp[0] = _pack16(z0, di8)
    zcp[1] = _pack16(z1, di8)
    dinv8[...] = di8


def _enc_call(x0, x1, d0, d1, w):
    grid = _GRID
    in_specs = [
        pl.BlockSpec((_RN, _D), lambda i: (i, 0)),
        pl.BlockSpec((_RN, _D), lambda i: (i, 0)),
        pl.BlockSpec((_RP, 8), lambda i: (i, 0)),
        pl.BlockSpec((_RP, 8), lambda i: (i, 0)),
    ] + [_full(a.shape) for a in w]
    return pl.pallas_call(
        _enc_body,
        grid=(grid,),
        in_specs=in_specs,
        out_specs=[pl.BlockSpec((_NC, _RP, 128), lambda i: (0, i, 0)),
                   pl.BlockSpec((_RP, 8), lambda i: (i, 0))],
        out_shape=[jax.ShapeDtypeStruct((_NC, _P, 128), jnp.float32),
                   jax.ShapeDtypeStruct((_P, 8), jnp.float32)],
    )(x0, x1, d0, d1, *w)


def _mid_body(Sp, dinv8, comb_W, comb_b, comb_g, comb_bb, clf_W1, clf_b1,
              clf_W2, clf_b2, z_out, zp_out, dom_out):
    di8 = dinv8[...]
    di = _unpack_col(di8)
    agg = di * jnp.concatenate([_unpack16(Sp[0]), _unpack16(Sp[1])], axis=1)
    t = jnp.dot(agg, comb_W[...], preferred_element_type=jnp.float32) + comb_b[...]
    z = _ln(t, comb_g[...], comb_bb[...])
    h = jnp.dot(z, clf_W1[...], preferred_element_type=jnp.float32) + clf_b1[...]
    h = _gelu(h)
    dom = jnp.dot(h, clf_W2[...], preferred_element_type=jnp.float32) + clf_b2[...]
    z_out[...] = z
    zp_out[...] = _pack16(z, di8)
    dom_out[...] = dom


def _mid_call(Sp, dinv8, w):
    grid = _GRID
    in_specs = [
        pl.BlockSpec((_NC, _RP, 128), lambda i: (0, i, 0)),
        pl.BlockSpec((_RP, 8), lambda i: (i, 0)),
    ] + [_full(a.shape) for a in w]
    return pl.pallas_call(
        _mid_body,
        grid=(grid,),
        in_specs=in_specs,
        out_specs=[pl.BlockSpec((_RN, _O), lambda i: (i, 0)),
                   pl.BlockSpec((_RP, 128), lambda i: (i, 0)),
                   pl.BlockSpec((_RN, 8), lambda i: (i, 0))],
        out_shape=[jax.ShapeDtypeStruct((_N, _O), jnp.float32),
                   jax.ShapeDtypeStruct((_P, 128), jnp.float32),
                   jax.ShapeDtypeStruct((_N, 8), jnp.float32)],
    )(Sp, dinv8, *w)


def _fin_body(S2p, zpp, dinv8, W0, b0, W1, b1, rec0, rec1):
    sum_p = S2p[0] + S2p[1] - zpp[...]
    di8 = dinv8[...]
    scale = jnp.concatenate(
        [jnp.broadcast_to(di8[:, s:s + 1], (_RP, _O)) for s in range(8)],
        axis=1)
    aggz = _unpack16(sum_p * scale)
    rec0[...] = jnp.dot(aggz, W0[...], preferred_element_type=jnp.float32) + b0[...]
    rec1[...] = jnp.dot(aggz, W1[...], preferred_element_type=jnp.float32) + b1[...]


def _fin_call(S2p, zpp, dinv8, w):
    grid = _GRID
    in_specs = [
        pl.BlockSpec((_NC, _RP, 128), lambda i: (0, i, 0)),
        pl.BlockSpec((_RP, 128), lambda i: (i, 0)),
        pl.BlockSpec((_RP, 8), lambda i: (i, 0)),
    ] + [_full(a.shape) for a in w]
    return pl.pallas_call(
        _fin_body,
        grid=(grid,),
        in_specs=in_specs,
        out_specs=[pl.BlockSpec((_RN, _D), lambda i: (i, 0)),
                   pl.BlockSpec((_RN, _D), lambda i: (i, 0))],
        out_shape=[jax.ShapeDtypeStruct((_N, _D), jnp.float32),
                   jax.ShapeDtypeStruct((_N, _D), jnp.float32)],
    )(S2p, zpp, dinv8, *w)


def kernel(x0, x1, edge_index, enc0_W1, enc0_b1, enc0_g1, enc0_bb1, enc0_W2,
           enc0_b2, enc0_rg, enc0_rb, enc1_W1, enc1_b1, enc1_g1, enc1_bb1,
           enc1_W2, enc1_b2, enc1_rg, enc1_rb, comb_W, comb_b, comb_g,
           comb_bb, dec0_W, dec0_b, dec1_W, dec1_b, clf_W1, clf_b1, clf_W2,
           clf_b2):
    f32 = jnp.float32
    ei3 = edge_index.reshape(2, _ER, 128)
    zcol = jnp.zeros((_N,), f32)
    ones128 = jnp.ones((128,), f32)

    deg0, deg1 = _deg_kernel(ei3, zcol, ones128)

    row = lambda a: a.reshape(1, -1)
    enc_w = (enc0_W1, row(enc0_b1), row(enc0_g1), row(enc0_bb1), enc0_W2,
             row(enc0_b2), row(enc0_rg), row(enc0_rb),
             enc1_W1, row(enc1_b1), row(enc1_g1), row(enc1_bb1), enc1_W2,
             row(enc1_b2), row(enc1_rg), row(enc1_rb))
    zcp_p, dinv8 = _enc_call(x0, x1, deg0.reshape(_P, 8), deg1.reshape(_P, 8),
                             enc_w)

    S = _agg1_kernel(zcp_p.reshape(_NC, _N, _O), ei3)

    mid_w = (comb_W, row(comb_b), row(comb_g), row(comb_bb),
             clf_W1, row(clf_b1), clf_W2, row(clf_b2))
    z, zp_p, dom = _mid_call(S.reshape(_NC, _P, 128), dinv8, mid_w)

    S2 = _agg2_kernel(zp_p.reshape(_N, _O), ei3)

    rec0, rec1 = _fin_call(S2.reshape(_NC, _P, 128), zp_p, dinv8,
                           (dec0_W, row(dec0_b), dec1_W, row(dec1_b)))
    return (z, rec0, rec1, dom)


# speedup vs baseline: 39.7755x; 1.3837x over previous
import functools

import jax
import jax.numpy as jnp
from jax import lax
from jax.experimental import pallas as pl
from jax.experimental.pallas import tpu as pltpu
from jax.experimental.pallas import tpu_sc as plsc

_N = 100000
_E = 1600000
_D = 128
_H = 64
_O = 16
_NC = 2
_NS = 16
_P = _N // 8
_ER = _E // 128
_RB = 4
_NCH = _ER // _RB
_C0_CH = 1562

_mesh = plsc.VectorSubcoreMesh(core_axis_name="c", subcore_axis_name="s",
                               num_cores=_NC, num_subcores=_NS)
_sc_params = pltpu.CompilerParams(use_tc_tiling_on_sc=False)

_OCT_Q, _OCT_R = divmod(_N // 8, _NS)


def _node_copy(sid, src_at, dst_at):
    base = pl.multiple_of((sid * _OCT_Q + jnp.minimum(sid, _OCT_R)) * 8, 8)
    pltpu.sync_copy(src_at(base, _OCT_Q * 8), dst_at(base, _OCT_Q * 8))

    @pl.when(sid < _OCT_R)
    def _():
        b1 = pl.multiple_of(base + _OCT_Q * 8, 8)
        pltpu.sync_copy(src_at(b1, 8), dst_at(b1, 8))


def _chunk_sched(nchunks, rm, sid):
    q = nchunks // _NS
    b = sid * q + jnp.minimum(sid, rm)
    n = q + jnp.where(sid < rm, 1, 0)
    return b, n


@functools.partial(
    pl.kernel,
    out_type=[jax.ShapeDtypeStruct((_N,), jnp.float32),
              jax.ShapeDtypeStruct((_N,), jnp.float32)],
    mesh=_mesh,
    compiler_params=_sc_params,
    scratch_types=[
        pltpu.VMEM((_RB, 128), jnp.int32),
        pltpu.VMEM((_RB, 128), jnp.int32),
        pltpu.VMEM((128,), jnp.float32),
        pltpu.VMEM_SHARED((_N,), jnp.float32),
        pltpu.SemaphoreType.DMA,
        pltpu.SemaphoreType.DMA,
    ],
)
def _deg_kernel(ei3_hbm, zcol_hbm, ones_hbm, out0_hbm, out1_hbm,
                didx0, didx1, ones_v, acc, sem0, sem1):
    cid = lax.axis_index("c")
    sid = lax.axis_index("s")
    pltpu.sync_copy(ones_hbm, ones_v)
    _node_copy(sid, lambda b, n: zcol_hbm.at[pl.ds(b, n)],
               lambda b, n: acc.at[pl.ds(b, n)])
    plsc.subcore_barrier()

    chunk0 = cid * _C0_CH
    b, n = _chunk_sched(_C0_CH, jnp.where(cid == 0, _C0_CH % _NS,
                                          (_NCH - _C0_CH) % _NS), sid)
    bufs = ((didx0, sem0), (didx1, sem1))

    def load(c, k):
        row = pl.multiple_of((chunk0 + b + c) * _RB, _RB)
        pltpu.sync_copy(ei3_hbm.at[1].at[pl.ds(row, _RB)], bufs[k][0])

    def fire(k):
        didx, sem = bufs[k]
        return [pltpu.async_copy(ones_v, acc.at[didx.at[j]], sem, add=True)
                for j in range(_RB)]

    def drain(k):
        didx, sem = bufs[k]
        for j in range(_RB):
            pltpu.make_async_copy(ones_v, acc.at[didx.at[j]], sem).wait()

    load(0, 0)
    fire(0)

    def pair(p, carry):
        c0 = 2 * p
        load(c0 + 1, 1)
        drain(0)
        fire(1)

        @pl.when(c0 + 2 < n)
        def _():
            load(c0 + 2, 0)
            fire(0)

        drain(1)
        return carry

    lax.fori_loop(0, n // 2, pair, 0)

    @pl.when(n % 2 == 1)
    def _():
        drain(0)

    plsc.subcore_barrier()

    @pl.when(cid == 0)
    def _():
        _node_copy(sid, lambda b2, n2: acc.at[pl.ds(b2, n2)],
                   lambda b2, n2: out0_hbm.at[pl.ds(b2, n2)])

    @pl.when(cid == 1)
    def _():
        _node_copy(sid, lambda b2, n2: acc.at[pl.ds(b2, n2)],
                   lambda b2, n2: out1_hbm.at[pl.ds(b2, n2)])


def _make_agg_kernel(per_core_features):
    @functools.partial(
        pl.kernel,
        out_type=[jax.ShapeDtypeStruct((_N, _O), jnp.float32),
                  jax.ShapeDtypeStruct((_N, _O), jnp.float32)],
        mesh=_mesh,
        compiler_params=_sc_params,
        scratch_types=[
            pltpu.VMEM((_RB, 128), jnp.int32),
            pltpu.VMEM((_RB, 128), jnp.int32),
            pltpu.VMEM((_RB, 128), jnp.int32),
            pltpu.VMEM((_RB, 128), jnp.int32),
            pltpu.VMEM((_RB * 128, _O), jnp.float32),
            pltpu.VMEM((_RB * 128, _O), jnp.float32),
            pltpu.VMEM_SHARED((_N, _O), jnp.float32),
            pltpu.SemaphoreType.DMA,
            pltpu.SemaphoreType.DMA,
            pltpu.SemaphoreType.DMA,
            pltpu.SemaphoreType.DMA,
        ],
    )
    def _agg(tab_hbm, ei3_hbm, out0_hbm, out1_hbm, sidx0, didx0, sidx1, didx1,
             rows0, rows1, acc, gs0, ss0, gs1, ss1):
        cid = lax.axis_index("c")
        sid = lax.axis_index("s")
        tab = tab_hbm.at[cid] if per_core_features else tab_hbm
        _node_copy(sid, lambda b, n: tab.at[pl.ds(b, n)],
                   lambda b, n: acc.at[pl.ds(b, n)])
        plsc.subcore_barrier()

        if per_core_features:
            chunk0 = 0
            b, n = _chunk_sched(_NCH, _NCH % _NS, sid)
        else:
            chunk0 = cid * _C0_CH
            b, n = _chunk_sched(_C0_CH, jnp.where(cid == 0, _C0_CH % _NS,
                                                  (_NCH - _C0_CH) % _NS), sid)

        bufs = ((sidx0, didx0, rows0, gs0, ss0),
                (sidx1, didx1, rows1, gs1, ss1))

        def load_fire(c, k):
            sidx, didx, rows, gsem, _ = bufs[k]
            row = pl.multiple_of((chunk0 + b + c) * _RB, _RB)
            pltpu.sync_copy(ei3_hbm.at[0].at[pl.ds(row, _RB)], sidx)
            pltpu.sync_copy(ei3_hbm.at[1].at[pl.ds(row, _RB)], didx)
            for j in range(_RB):
                pltpu.async_copy(tab.at[sidx.at[j]],
                                 rows.at[pl.ds(j * 128, 128)], gsem)

        def scatter(k):
            sidx, didx, rows, gsem, ssem = bufs[k]
            for j in range(_RB):
                pltpu.make_async_copy(tab.at[sidx.at[j]],
                                      rows.at[pl.ds(j * 128, 128)],
                                      gsem).wait()
            for j in range(_RB):
                pltpu.async_copy(rows.at[pl.ds(j * 128, 128)],
                                 acc.at[didx.at[j]], ssem, add=True)

        def drain(k):
            sidx, didx, rows, _, ssem = bufs[k]
            for j in range(_RB):
                pltpu.make_async_copy(rows.at[pl.ds(j * 128, 128)],
                                      acc.at[didx.at[j]], ssem).wait()

        load_fire(0, 0)

        def pair(p, carry):
            c0 = 2 * p
            load_fire(c0 + 1, 1)
            scatter(0)
            drain(0)

            @pl.when(c0 + 2 < n)
            def _():
                load_fire(c0 + 2, 0)

            scatter(1)
            drain(1)
            return carry

        lax.fori_loop(0, n // 2, pair, 0)

        @pl.when(n % 2 == 1)
        def _():
            scatter(0)
            drain(0)

        plsc.subcore_barrier()

        @pl.when(cid == 0)
        def _():
            _node_copy(sid, lambda b2, n2: acc.at[pl.ds(b2, n2)],
                       lambda b2, n2: out0_hbm.at[pl.ds(b2, n2)])

        @pl.when(cid == 1)
        def _():
            _node_copy(sid, lambda b2, n2: acc.at[pl.ds(b2, n2)],
                       lambda b2, n2: out1_hbm.at[pl.ds(b2, n2)])

    return _agg


_agg1_kernel = _make_agg_kernel(True)
_agg2_kernel = _make_agg_kernel(False)


def _gelu(x):
    return 0.5 * x * (1.0 + lax.erf(x * 0.7071067811865476))


def _ln(x, g, b, eps=1e-5):
    m = jnp.mean(x, axis=-1, keepdims=True)
    v = jnp.mean((x - m) ** 2, axis=-1, keepdims=True)
    return (x - m) / jnp.sqrt(v + eps) * g + b


_RN = 1984
_RP = _RN // 8
_GRID = (_N + _RN - 1) // _RN


def _pack16(z, scale8=None):
    z3 = z.reshape(_RP, 8, _O)
    parts = []
    for s in range(8):
        p = z3[:, s, :]
        if scale8 is not None:
            p = p * scale8[:, s:s + 1]
        parts.append(p)
    return jnp.concatenate(parts, axis=1)


def _unpack16(zp):
    parts = [zp[:, 16 * s:16 * (s + 1)].reshape(_RP, 1, _O) for s in range(8)]
    return jnp.concatenate(parts, axis=1).reshape(_RN, _O)


def _full(shape):
    return pl.BlockSpec(shape, lambda i: tuple(0 for _ in shape))


def _enc_body(x0, x1, d0, d1, W10, b10, g10, bb10, W20, b20, rg0, rb0,
              W11, b11, g11, bb11, W21, b21, rg1, rb1, zcp, dinv8):
    di8 = lax.rsqrt(d0[...] + d1[...] + 1.0)

    def enc(x, W1, b1, g1, bb1, W2, b2, rg, rb):
        h = jnp.dot(x, W1, preferred_element_type=jnp.float32) + b1
        h = _ln(h, g1, bb1)
        h = _gelu(h)
        h2 = jnp.dot(h, W2, preferred_element_type=jnp.float32) + b2
        t = h2 + h2
        return _ln(t, rg, rb)

    z0 = enc(x0[...], W10[...], b10[...], g10[...], bb10[...],
             W20[...], b20[...], rg0[...], rb0[...])
    z1 = enc(x1[...], W11[...], b11[...], g11[...], bb11[...],
             W21[...], b21[...], rg1[...], rb1[...])
    zcp[0] = _pack16(z0, di8)
    zcp[1] = _pack16(z1, di8)
    dinv8[...] = di8


def _enc_call(x0, x1, d0, d1, w):
    grid = _GRID
    in_specs = [
        pl.BlockSpec((_RN, _D), lambda i: (i, 0)),
        pl.BlockSpec((_RN, _D), lambda i: (i, 0)),
        pl.BlockSpec((_RP, 8), lambda i: (i, 0)),
        pl.BlockSpec((_RP, 8), lambda i: (i, 0)),
    ] + [_full(a.shape) for a in w]
    return pl.pallas_call(
        _enc_body,
        grid=(grid,),
        in_specs=in_specs,
        out_specs=[pl.BlockSpec((_NC, _RP, 128), lambda i: (0, i, 0)),
                   pl.BlockSpec((_RP, 8), lambda i: (i, 0))],
        out_shape=[jax.ShapeDtypeStruct((_NC, _P, 128), jnp.float32),
                   jax.ShapeDtypeStruct((_P, 8), jnp.float32)],
    )(x0, x1, d0, d1, *w)


def _mid_body(S0p, S1p, dinv8, dsel, combk0, combk1, combb, kJ, g128, bb128,
              clfW1k, clfb1, clfW2k, clfb2, zp_pk, zpp_out, domp_out):
    dscale = jnp.dot(dinv8[...], dsel[...],
                     preferred_element_type=jnp.float32)
    t = (jnp.dot(S0p[...] * dscale, combk0[...],
                 preferred_element_type=jnp.float32)
         + jnp.dot(S1p[...] * dscale, combk1[...],
                   preferred_element_type=jnp.float32) + combb[...])
    m = jnp.dot(t, kJ[...], preferred_element_type=jnp.float32)
    c = t - m
    v = jnp.dot(c * c, kJ[...], preferred_element_type=jnp.float32)
    zP = c * lax.rsqrt(v + 1e-5) * g128[...] + bb128[...]
    h = _gelu(jnp.dot(zP, clfW1k[...], preferred_element_type=jnp.float32)
              + clfb1[...])
    domP = jnp.dot(h, clfW2k[...], preferred_element_type=jnp.float32) \
        + clfb2[...]
    zp_pk[...] = zP
    zpp_out[...] = zP * dscale
    domp_out[...] = domP


def _mid_call(S0p, S1p, dinv8, w):
    grid = _GRID
    in_specs = [
        pl.BlockSpec((_RP, 128), lambda i: (i, 0)),
        pl.BlockSpec((_RP, 128), lambda i: (i, 0)),
        pl.BlockSpec((_RP, 8), lambda i: (i, 0)),
    ] + [_full(a.shape) for a in w]
    return pl.pallas_call(
        _mid_body,
        grid=(grid,),
        in_specs=in_specs,
        out_specs=[pl.BlockSpec((_RP, 128), lambda i: (i, 0)),
                   pl.BlockSpec((_RP, 128), lambda i: (i, 0)),
                   pl.BlockSpec((_RP, 64), lambda i: (i, 0))],
        out_shape=[jax.ShapeDtypeStruct((_P, 128), jnp.float32),
                   jax.ShapeDtypeStruct((_P, 128), jnp.float32),
                   jax.ShapeDtypeStruct((_P, 64), jnp.float32)],
    )(S0p, S1p, dinv8, *w)


def _fin_body(S2a, S2b, zpp, dinv8, dsel, W0, b0, W1, b1, rec0, rec1):
    dscale = jnp.dot(dinv8[...], dsel[...],
                     preferred_element_type=jnp.float32)
    sum_p = (S2a[...] + S2b[...] - zpp[...]) * dscale
    aggz = _unpack16(sum_p)
    rec0[...] = jnp.dot(aggz, W0[...], preferred_element_type=jnp.float32) + b0[...]
    rec1[...] = jnp.dot(aggz, W1[...], preferred_element_type=jnp.float32) + b1[...]


def _fin_call(S2a, S2b, zpp, dinv8, w):
    grid = _GRID
    in_specs = [
        pl.BlockSpec((_RP, 128), lambda i: (i, 0)),
        pl.BlockSpec((_RP, 128), lambda i: (i, 0)),
        pl.BlockSpec((_RP, 128), lambda i: (i, 0)),
        pl.BlockSpec((_RP, 8), lambda i: (i, 0)),
    ] + [_full(a.shape) for a in w]
    return pl.pallas_call(
        _fin_body,
        grid=(grid,),
        in_specs=in_specs,
        out_specs=[pl.BlockSpec((_RN, _D), lambda i: (i, 0)),
                   pl.BlockSpec((_RN, _D), lambda i: (i, 0))],
        out_shape=[jax.ShapeDtypeStruct((_N, _D), jnp.float32),
                   jax.ShapeDtypeStruct((_N, _D), jnp.float32)],
    )(S2a, S2b, zpp, dinv8, *w)


def kernel(x0, x1, edge_index, enc0_W1, enc0_b1, enc0_g1, enc0_bb1, enc0_W2,
           enc0_b2, enc0_rg, enc0_rb, enc1_W1, enc1_b1, enc1_g1, enc1_bb1,
           enc1_W2, enc1_b2, enc1_rg, enc1_rb, comb_W, comb_b, comb_g,
           comb_bb, dec0_W, dec0_b, dec1_W, dec1_b, clf_W1, clf_b1, clf_W2,
           clf_b2):
    f32 = jnp.float32
    ei3 = edge_index.reshape(2, _ER, 128)
    zcol = jnp.zeros((_N,), f32)
    ones128 = jnp.ones((128,), f32)

    deg0, deg1 = _deg_kernel(ei3, zcol, ones128)

    row = lambda a: a.reshape(1, -1)
    enc_w = (enc0_W1, row(enc0_b1), row(enc0_g1), row(enc0_bb1), enc0_W2,
             row(enc0_b2), row(enc0_rg), row(enc0_rb),
             enc1_W1, row(enc1_b1), row(enc1_g1), row(enc1_bb1), enc1_W2,
             row(enc1_b2), row(enc1_rg), row(enc1_rb))
    zcp_p, dinv8 = _enc_call(x0, x1, deg0.reshape(_P, 8), deg1.reshape(_P, 8),
                             enc_w)

    S0, S1 = _agg1_kernel(zcp_p.reshape(_NC, _N, _O), ei3)

    eye8 = jnp.eye(8, dtype=f32)
    kron8 = lambda W: jnp.kron(eye8, W)
    tile8 = lambda v: jnp.tile(v, 8).reshape(1, -1)
    dsel = kron8(jnp.ones((1, _O), f32))
    mid_w = (dsel, kron8(comb_W[:_O]), kron8(comb_W[_O:]), tile8(comb_b),
             kron8(jnp.full((_O, _O), 1.0 / _O, f32)), tile8(comb_g),
             tile8(comb_bb), kron8(clf_W1), tile8(clf_b1), kron8(clf_W2),
             tile8(clf_b2))
    zP, zpP, domP = _mid_call(S0.reshape(_P, 128), S1.reshape(_P, 128),
                              dinv8, mid_w)

    S2a, S2b = _agg2_kernel(zpP.reshape(_N, _O), ei3)

    rec0, rec1 = _fin_call(S2a.reshape(_P, 128), S2b.reshape(_P, 128),
                           zpP, dinv8,
                           (dsel, dec0_W, row(dec0_b), dec1_W, row(dec1_b)))
    return (zP.reshape(_N, _O), rec0, rec1, domP.reshape(_N, 8))


# speedup vs baseline: 43.9008x; 1.1037x over previous
import functools

import jax
import jax.numpy as jnp
from jax import lax
from jax.experimental import pallas as pl
from jax.experimental.pallas import tpu as pltpu
from jax.experimental.pallas import tpu_sc as plsc

_N = 100000
_E = 1600000
_D = 128
_H = 64
_O = 16
_NC = 2
_NS = 16
_P = _N // 8
_ER = _E // 128
_RB = 4
_NCH = _ER // _RB
_C0_CH = 1562

_mesh = plsc.VectorSubcoreMesh(core_axis_name="c", subcore_axis_name="s",
                               num_cores=_NC, num_subcores=_NS)
_sc_params = pltpu.CompilerParams(use_tc_tiling_on_sc=False)

_OCT_Q, _OCT_R = divmod(_N // 8, _NS)


def _node_copy(sid, src_at, dst_at):
    base = pl.multiple_of((sid * _OCT_Q + jnp.minimum(sid, _OCT_R)) * 8, 8)
    pltpu.sync_copy(src_at(base, _OCT_Q * 8), dst_at(base, _OCT_Q * 8))

    @pl.when(sid < _OCT_R)
    def _():
        b1 = pl.multiple_of(base + _OCT_Q * 8, 8)
        pltpu.sync_copy(src_at(b1, 8), dst_at(b1, 8))


def _chunk_sched(nchunks, rm, sid):
    q = nchunks // _NS
    b = sid * q + jnp.minimum(sid, rm)
    n = q + jnp.where(sid < rm, 1, 0)
    return b, n


@functools.partial(
    pl.kernel,
    out_type=[jax.ShapeDtypeStruct((_N,), jnp.float32),
              jax.ShapeDtypeStruct((_N,), jnp.float32)],
    mesh=_mesh,
    compiler_params=_sc_params,
    scratch_types=[
        pltpu.VMEM((_RB, 128), jnp.int32),
        pltpu.VMEM((_RB, 128), jnp.int32),
        pltpu.VMEM((128,), jnp.float32),
        pltpu.VMEM_SHARED((_N,), jnp.float32),
        pltpu.SemaphoreType.DMA,
        pltpu.SemaphoreType.DMA,
    ],
)
def _deg_kernel(ei3_hbm, zcol_hbm, ones_hbm, out0_hbm, out1_hbm,
                didx0, didx1, ones_v, acc, sem0, sem1):
    cid = lax.axis_index("c")
    sid = lax.axis_index("s")
    pltpu.sync_copy(ones_hbm, ones_v)
    _node_copy(sid, lambda b, n: zcol_hbm.at[pl.ds(b, n)],
               lambda b, n: acc.at[pl.ds(b, n)])
    plsc.subcore_barrier()

    chunk0 = cid * _C0_CH
    b, n = _chunk_sched(_C0_CH, jnp.where(cid == 0, _C0_CH % _NS,
                                          (_NCH - _C0_CH) % _NS), sid)
    bufs = ((didx0, sem0), (didx1, sem1))

    def load(c, k):
        row = pl.multiple_of((chunk0 + b + c) * _RB, _RB)
        pltpu.sync_copy(ei3_hbm.at[1].at[pl.ds(row, _RB)], bufs[k][0])

    def fire(k):
        didx, sem = bufs[k]
        return [pltpu.async_copy(ones_v, acc.at[didx.at[j]], sem, add=True)
                for j in range(_RB)]

    def drain(k):
        didx, sem = bufs[k]
        for j in range(_RB):
            pltpu.make_async_copy(ones_v, acc.at[didx.at[j]], sem).wait()

    load(0, 0)
    fire(0)

    def pair(p, carry):
        c0 = 2 * p
        load(c0 + 1, 1)
        drain(0)
        fire(1)

        @pl.when(c0 + 2 < n)
        def _():
            load(c0 + 2, 0)
            fire(0)

        drain(1)
        return carry

    lax.fori_loop(0, n // 2, pair, 0)

    @pl.when(n % 2 == 1)
    def _():
        drain(0)

    plsc.subcore_barrier()

    @pl.when(cid == 0)
    def _():
        _node_copy(sid, lambda b2, n2: acc.at[pl.ds(b2, n2)],
                   lambda b2, n2: out0_hbm.at[pl.ds(b2, n2)])

    @pl.when(cid == 1)
    def _():
        _node_copy(sid, lambda b2, n2: acc.at[pl.ds(b2, n2)],
                   lambda b2, n2: out1_hbm.at[pl.ds(b2, n2)])


def _make_agg_kernel(per_core_features):
    @functools.partial(
        pl.kernel,
        out_type=[jax.ShapeDtypeStruct((_N, _O), jnp.float32),
                  jax.ShapeDtypeStruct((_N, _O), jnp.float32)],
        mesh=_mesh,
        compiler_params=_sc_params,
        scratch_types=[
            pltpu.VMEM((_RB, 128), jnp.int32),
            pltpu.VMEM((_RB, 128), jnp.int32),
            pltpu.VMEM((_RB, 128), jnp.int32),
            pltpu.VMEM((_RB, 128), jnp.int32),
            pltpu.VMEM((_RB * 128, _O), jnp.float32),
            pltpu.VMEM((_RB * 128, _O), jnp.float32),
            pltpu.VMEM_SHARED((_N, _O), jnp.float32),
            pltpu.SemaphoreType.DMA,
            pltpu.SemaphoreType.DMA,
            pltpu.SemaphoreType.DMA,
            pltpu.SemaphoreType.DMA,
        ],
    )
    def _agg(tab_hbm, ei3_hbm, out0_hbm, out1_hbm, sidx0, didx0, sidx1, didx1,
             rows0, rows1, acc, gs0, ss0, gs1, ss1):
        cid = lax.axis_index("c")
        sid = lax.axis_index("s")
        tab = tab_hbm.at[cid] if per_core_features else tab_hbm
        _node_copy(sid, lambda b, n: tab.at[pl.ds(b, n)],
                   lambda b, n: acc.at[pl.ds(b, n)])
        plsc.subcore_barrier()

        if per_core_features:
            chunk0 = 0
            b, n = _chunk_sched(_NCH, _NCH % _NS, sid)
        else:
            chunk0 = cid * _C0_CH
            b, n = _chunk_sched(_C0_CH, jnp.where(cid == 0, _C0_CH % _NS,
                                                  (_NCH - _C0_CH) % _NS), sid)

        bufs = ((sidx0, didx0, rows0, gs0, ss0),
                (sidx1, didx1, rows1, gs1, ss1))

        def load_fire(c, k):
            sidx, didx, rows, gsem, _ = bufs[k]
            row = pl.multiple_of((chunk0 + b + c) * _RB, _RB)
            pltpu.sync_copy(ei3_hbm.at[0].at[pl.ds(row, _RB)], sidx)
            pltpu.sync_copy(ei3_hbm.at[1].at[pl.ds(row, _RB)], didx)
            for j in range(_RB):
                pltpu.async_copy(tab.at[sidx.at[j]],
                                 rows.at[pl.ds(j * 128, 128)], gsem)

        def scatter(k):
            sidx, didx, rows, gsem, ssem = bufs[k]
            for j in range(_RB):
                pltpu.make_async_copy(tab.at[sidx.at[j]],
                                      rows.at[pl.ds(j * 128, 128)],
                                      gsem).wait()
            for j in range(_RB):
                pltpu.async_copy(rows.at[pl.ds(j * 128, 128)],
                                 acc.at[didx.at[j]], ssem, add=True)

        def drain(k):
            sidx, didx, rows, _, ssem = bufs[k]
            for j in range(_RB):
                pltpu.make_async_copy(rows.at[pl.ds(j * 128, 128)],
                                      acc.at[didx.at[j]], ssem).wait()

        load_fire(0, 0)

        def pair(p, carry):
            c0 = 2 * p
            load_fire(c0 + 1, 1)
            scatter(0)
            drain(0)

            @pl.when(c0 + 2 < n)
            def _():
                load_fire(c0 + 2, 0)

            scatter(1)
            drain(1)
            return carry

        lax.fori_loop(0, n // 2, pair, 0)

        @pl.when(n % 2 == 1)
        def _():
            scatter(0)
            drain(0)

        plsc.subcore_barrier()

        @pl.when(cid == 0)
        def _():
            _node_copy(sid, lambda b2, n2: acc.at[pl.ds(b2, n2)],
                       lambda b2, n2: out0_hbm.at[pl.ds(b2, n2)])

        @pl.when(cid == 1)
        def _():
            _node_copy(sid, lambda b2, n2: acc.at[pl.ds(b2, n2)],
                       lambda b2, n2: out1_hbm.at[pl.ds(b2, n2)])

    return _agg


_agg1_kernel = _make_agg_kernel(True)
_agg2_kernel = _make_agg_kernel(False)


def _gelu(x):
    return 0.5 * x * (1.0 + lax.erf(x * 0.7071067811865476))


def _ln(x, g, b, eps=1e-5):
    m = jnp.mean(x, axis=-1, keepdims=True)
    v = jnp.mean((x - m) ** 2, axis=-1, keepdims=True)
    return (x - m) / jnp.sqrt(v + eps) * g + b


_RN = 1984
_RP = _RN // 8
_GRID = (_N + _RN - 1) // _RN


def _pack16(z, scale8=None):
    z3 = z.reshape(_RP, 8, _O)
    parts = []
    for s in range(8):
        p = z3[:, s, :]
        if scale8 is not None:
            p = p * scale8[:, s:s + 1]
        parts.append(p)
    return jnp.concatenate(parts, axis=1)


def _unpack16(zp):
    parts = [zp[:, 16 * s:16 * (s + 1)].reshape(_RP, 1, _O) for s in range(8)]
    return jnp.concatenate(parts, axis=1).reshape(_RN, _O)


def _full(shape):
    return pl.BlockSpec(shape, lambda i: tuple(0 for _ in shape))


def _enc_body(x0, x1, d0, d1, W10, b10, g10, bb10, W11, b11, g11, bb11,
              Wb0, b2t0, rg0, rb0, Wb1, b2t1, rg1, rb1, dsel, kJ,
              zcp, dinv8):
    di8 = lax.rsqrt(d0[...] + d1[...] + 1.0)
    dscale = jnp.dot(di8, dsel[...], preferred_element_type=jnp.float32)

    def hidden(x, W1, b1, g1, bb1):
        h = jnp.dot(x, W1, preferred_element_type=jnp.float32) + b1
        return _gelu(_ln(h, g1, bb1))

    h0 = hidden(x0[...], W10[...], b10[...], g10[...], bb10[...])
    h1 = hidden(x1[...], W11[...], b11[...], g11[...], bb11[...])
    h3 = jnp.concatenate([h0, h1], axis=1).reshape(_RP, 8, _D)
    hP = jnp.concatenate([h3[:, s, :] for s in range(8)], axis=1)

    def head(Wb, b2t, rg, rb):
        h2 = jnp.dot(hP, Wb[...], preferred_element_type=jnp.float32) + b2t[...]
        t = h2 + h2
        m = jnp.dot(t, kJ[...], preferred_element_type=jnp.float32)
        c = t - m
        v = jnp.dot(c * c, kJ[...], preferred_element_type=jnp.float32)
        return (c * lax.rsqrt(v + 1e-5) * rg[...] + rb[...]) * dscale

    zcp[0] = head(Wb0, b2t0, rg0, rb0)
    zcp[1] = head(Wb1, b2t1, rg1, rb1)
    dinv8[...] = di8


def _enc_call(x0, x1, d0, d1, w):
    grid = _GRID
    in_specs = [
        pl.BlockSpec((_RN, _D), lambda i: (i, 0)),
        pl.BlockSpec((_RN, _D), lambda i: (i, 0)),
        pl.BlockSpec((_RP, 8), lambda i: (i, 0)),
        pl.BlockSpec((_RP, 8), lambda i: (i, 0)),
    ] + [_full(a.shape) for a in w]
    return pl.pallas_call(
        _enc_body,
        grid=(grid,),
        in_specs=in_specs,
        out_specs=[pl.BlockSpec((_NC, _RP, 128), lambda i: (0, i, 0)),
                   pl.BlockSpec((_RP, 8), lambda i: (i, 0))],
        out_shape=[jax.ShapeDtypeStruct((_NC, _P, 128), jnp.float32),
                   jax.ShapeDtypeStruct((_P, 8), jnp.float32)],
    )(x0, x1, d0, d1, *w)


def _mid_body(S0p, S1p, dinv8, dsel, combk0, combk1, combb, kJ, g128, bb128,
              clfW1k, clfb1, clfW2k, clfb2, zp_pk, zpp_out, domp_out):
    dscale = jnp.dot(dinv8[...], dsel[...],
                     preferred_element_type=jnp.float32)
    t = (jnp.dot(S0p[...] * dscale, combk0[...],
                 preferred_element_type=jnp.float32)
         + jnp.dot(S1p[...] * dscale, combk1[...],
                   preferred_element_type=jnp.float32) + combb[...])
    m = jnp.dot(t, kJ[...], preferred_element_type=jnp.float32)
    c = t - m
    v = jnp.dot(c * c, kJ[...], preferred_element_type=jnp.float32)
    zP = c * lax.rsqrt(v + 1e-5) * g128[...] + bb128[...]
    h = _gelu(jnp.dot(zP, clfW1k[...], preferred_element_type=jnp.float32)
              + clfb1[...])
    domP = jnp.dot(h, clfW2k[...], preferred_element_type=jnp.float32) \
        + clfb2[...]
    zp_pk[...] = zP
    zpp_out[...] = zP * dscale
    domp_out[...] = domP


def _mid_call(S0p, S1p, dinv8, w):
    grid = _GRID
    in_specs = [
        pl.BlockSpec((_RP, 128), lambda i: (i, 0)),
        pl.BlockSpec((_RP, 128), lambda i: (i, 0)),
        pl.BlockSpec((_RP, 8), lambda i: (i, 0)),
    ] + [_full(a.shape) for a in w]
    return pl.pallas_call(
        _mid_body,
        grid=(grid,),
        in_specs=in_specs,
        out_specs=[pl.BlockSpec((_RP, 128), lambda i: (i, 0)),
                   pl.BlockSpec((_RP, 128), lambda i: (i, 0)),
                   pl.BlockSpec((_RP, 64), lambda i: (i, 0))],
        out_shape=[jax.ShapeDtypeStruct((_P, 128), jnp.float32),
                   jax.ShapeDtypeStruct((_P, 128), jnp.float32),
                   jax.ShapeDtypeStruct((_P, 64), jnp.float32)],
    )(S0p, S1p, dinv8, *w)


def _fin_body(S2a, S2b, zpp, dinv8, dsel, W0, b0, W1, b1, rec0, rec1):
    dscale = jnp.dot(dinv8[...], dsel[...],
                     preferred_element_type=jnp.float32)
    sum_p = (S2a[...] + S2b[...] - zpp[...]) * dscale
    aggz = _unpack16(sum_p)
    rec0[...] = jnp.dot(aggz, W0[...], preferred_element_type=jnp.float32) + b0[...]
    rec1[...] = jnp.dot(aggz, W1[...], preferred_element_type=jnp.float32) + b1[...]


def _fin_call(S2a, S2b, zpp, dinv8, w):
    grid = _GRID
    in_specs = [
        pl.BlockSpec((_RP, 128), lambda i: (i, 0)),
        pl.BlockSpec((_RP, 128), lambda i: (i, 0)),
        pl.BlockSpec((_RP, 128), lambda i: (i, 0)),
        pl.BlockSpec((_RP, 8), lambda i: (i, 0)),
    ] + [_full(a.shape) for a in w]
    return pl.pallas_call(
        _fin_body,
        grid=(grid,),
        in_specs=in_specs,
        out_specs=[pl.BlockSpec((_RN, _D), lambda i: (i, 0)),
                   pl.BlockSpec((_RN, _D), lambda i: (i, 0))],
        out_shape=[jax.ShapeDtypeStruct((_N, _D), jnp.float32),
                   jax.ShapeDtypeStruct((_N, _D), jnp.float32)],
    )(S2a, S2b, zpp, dinv8, *w)


def kernel(x0, x1, edge_index, enc0_W1, enc0_b1, enc0_g1, enc0_bb1, enc0_W2,
           enc0_b2, enc0_rg, enc0_rb, enc1_W1, enc1_b1, enc1_g1, enc1_bb1,
           enc1_W2, enc1_b2, enc1_rg, enc1_rb, comb_W, comb_b, comb_g,
           comb_bb, dec0_W, dec0_b, dec1_W, dec1_b, clf_W1, clf_b1, clf_W2,
           clf_b2):
    f32 = jnp.float32
    ei3 = edge_index.reshape(2, _ER, 128)
    zcol = jnp.zeros((_N,), f32)
    ones128 = jnp.ones((128,), f32)

    deg0, deg1 = _deg_kernel(ei3, zcol, ones128)

    row = lambda a: a.reshape(1, -1)
    eye8 = jnp.eye(8, dtype=f32)
    kron8 = lambda W: jnp.kron(eye8, W)
    tile8 = lambda v: jnp.tile(v, 8).reshape(1, -1)
    dsel = kron8(jnp.ones((1, _O), f32))
    kJ = kron8(jnp.full((_O, _O), 1.0 / _O, f32))
    z64 = jnp.zeros((_H, _O), f32)
    enc_w = (enc0_W1, row(enc0_b1), row(enc0_g1), row(enc0_bb1),
             enc1_W1, row(enc1_b1), row(enc1_g1), row(enc1_bb1),
             kron8(jnp.concatenate([enc0_W2, z64], axis=0)), tile8(enc0_b2),
             tile8(enc0_rg), tile8(enc0_rb),
             kron8(jnp.concatenate([z64, enc1_W2], axis=0)), tile8(enc1_b2),
             tile8(enc1_rg), tile8(enc1_rb), dsel, kJ)
    zcp_p, dinv8 = _enc_call(x0, x1, deg0.reshape(_P, 8), deg1.reshape(_P, 8),
                             enc_w)

    S0, S1 = _agg1_kernel(zcp_p.reshape(_NC, _N, _O), ei3)

    mid_w = (dsel, kron8(comb_W[:_O]), kron8(comb_W[_O:]), tile8(comb_b),
             kJ, tile8(comb_g), tile8(comb_bb), kron8(clf_W1), tile8(clf_b1),
             kron8(clf_W2), tile8(clf_b2))
    zP, zpP, domP = _mid_call(S0.reshape(_P, 128), S1.reshape(_P, 128),
                              dinv8, mid_w)

    S2a, S2b = _agg2_kernel(zpP.reshape(_N, _O), ei3)

    rec0, rec1 = _fin_call(S2a.reshape(_P, 128), S2b.reshape(_P, 128),
                           zpP, dinv8,
                           (dsel, dec0_W, row(dec0_b), dec1_W, row(dec1_b)))
    return (zP.reshape(_N, _O), rec0, rec1, domP.reshape(_N, 8))


# speedup vs baseline: 49.0582x; 1.1175x over previous
import functools

import jax
import jax.numpy as jnp
from jax import lax
from jax.experimental import pallas as pl
from jax.experimental.pallas import tpu as pltpu
from jax.experimental.pallas import tpu_sc as plsc

_N = 100000
_E = 1600000
_D = 128
_H = 64
_O = 16
_NC = 2
_NS = 16
_P = _N // 8
_ER = _E // 128
_RB = 4
_NCH = _ER // _RB
_C0_CH = 1562

_mesh = plsc.VectorSubcoreMesh(core_axis_name="c", subcore_axis_name="s",
                               num_cores=_NC, num_subcores=_NS)
_sc_params = pltpu.CompilerParams(use_tc_tiling_on_sc=False)

_OCT_Q, _OCT_R = divmod(_N // 8, _NS)


def _node_copy(sid, src_at, dst_at):
    base = pl.multiple_of((sid * _OCT_Q + jnp.minimum(sid, _OCT_R)) * 8, 8)
    pltpu.sync_copy(src_at(base, _OCT_Q * 8), dst_at(base, _OCT_Q * 8))

    @pl.when(sid < _OCT_R)
    def _():
        b1 = pl.multiple_of(base + _OCT_Q * 8, 8)
        pltpu.sync_copy(src_at(b1, 8), dst_at(b1, 8))


def _chunk_sched(nchunks, rm, sid):
    q = nchunks // _NS
    b = sid * q + jnp.minimum(sid, rm)
    n = q + jnp.where(sid < rm, 1, 0)
    return b, n


@functools.partial(
    pl.kernel,
    out_type=[jax.ShapeDtypeStruct((_N,), jnp.float32),
              jax.ShapeDtypeStruct((_N,), jnp.float32)],
    mesh=_mesh,
    compiler_params=_sc_params,
    scratch_types=[
        pltpu.VMEM((_RB, 128), jnp.int32),
        pltpu.VMEM((_RB, 128), jnp.int32),
        pltpu.VMEM((128,), jnp.float32),
        pltpu.VMEM_SHARED((_N,), jnp.float32),
        pltpu.SemaphoreType.DMA,
        pltpu.SemaphoreType.DMA,
    ],
)
def _deg_kernel(ei3_hbm, zcol_hbm, ones_hbm, out0_hbm, out1_hbm,
                didx0, didx1, ones_v, acc, sem0, sem1):
    cid = lax.axis_index("c")
    sid = lax.axis_index("s")
    pltpu.sync_copy(ones_hbm, ones_v)
    _node_copy(sid, lambda b, n: zcol_hbm.at[pl.ds(b, n)],
               lambda b, n: acc.at[pl.ds(b, n)])
    plsc.subcore_barrier()

    chunk0 = cid * _C0_CH
    b, n = _chunk_sched(_C0_CH, jnp.where(cid == 0, _C0_CH % _NS,
                                          (_NCH - _C0_CH) % _NS), sid)
    bufs = ((didx0, sem0), (didx1, sem1))

    def load(c, k):
        row = pl.multiple_of((chunk0 + b + c) * _RB, _RB)
        pltpu.sync_copy(ei3_hbm.at[1].at[pl.ds(row, _RB)], bufs[k][0])

    def fire(k):
        didx, sem = bufs[k]
        return [pltpu.async_copy(ones_v, acc.at[didx.at[j]], sem, add=True)
                for j in range(_RB)]

    def drain(k):
        didx, sem = bufs[k]
        for j in range(_RB):
            pltpu.make_async_copy(ones_v, acc.at[didx.at[j]], sem).wait()

    load(0, 0)
    fire(0)

    def pair(p, carry):
        c0 = 2 * p
        load(c0 + 1, 1)
        drain(0)
        fire(1)

        @pl.when(c0 + 2 < n)
        def _():
            load(c0 + 2, 0)
            fire(0)

        drain(1)
        return carry

    lax.fori_loop(0, n // 2, pair, 0)

    @pl.when(n % 2 == 1)
    def _():
        drain(0)

    plsc.subcore_barrier()

    @pl.when(cid == 0)
    def _():
        _node_copy(sid, lambda b2, n2: acc.at[pl.ds(b2, n2)],
                   lambda b2, n2: out0_hbm.at[pl.ds(b2, n2)])

    @pl.when(cid == 1)
    def _():
        _node_copy(sid, lambda b2, n2: acc.at[pl.ds(b2, n2)],
                   lambda b2, n2: out1_hbm.at[pl.ds(b2, n2)])


def _make_agg_kernel(per_core_features):
    @functools.partial(
        pl.kernel,
        out_type=[jax.ShapeDtypeStruct((_N, _O), jnp.float32),
                  jax.ShapeDtypeStruct((_N, _O), jnp.float32)],
        mesh=_mesh,
        compiler_params=_sc_params,
        scratch_types=[
            pltpu.VMEM((_RB, 128), jnp.int32),
            pltpu.VMEM((_RB, 128), jnp.int32),
            pltpu.VMEM((_RB, 128), jnp.int32),
            pltpu.VMEM((_RB, 128), jnp.int32),
            pltpu.VMEM((_RB * 128, _O), jnp.float32),
            pltpu.VMEM((_RB * 128, _O), jnp.float32),
            pltpu.VMEM_SHARED((_N, _O), jnp.float32),
            pltpu.SemaphoreType.DMA,
            pltpu.SemaphoreType.DMA,
            pltpu.SemaphoreType.DMA,
            pltpu.SemaphoreType.DMA,
            pltpu.SemaphoreType.DMA,
            pltpu.SemaphoreType.DMA,
        ],
    )
    def _agg(tab_hbm, ei3_hbm, out0_hbm, out1_hbm, sidx0, didx0, sidx1, didx1,
             rows0, rows1, acc, gs0, ss0, gs1, ss1, is0, is1):
        cid = lax.axis_index("c")
        sid = lax.axis_index("s")
        tab = tab_hbm.at[cid] if per_core_features else tab_hbm
        _node_copy(sid, lambda b, n: tab.at[pl.ds(b, n)],
                   lambda b, n: acc.at[pl.ds(b, n)])
        plsc.subcore_barrier()

        if per_core_features:
            chunk0 = 0
            b, n = _chunk_sched(_NCH, _NCH % _NS, sid)
        else:
            chunk0 = cid * _C0_CH
            b, n = _chunk_sched(_C0_CH, jnp.where(cid == 0, _C0_CH % _NS,
                                                  (_NCH - _C0_CH) % _NS), sid)

        bufs = ((sidx0, didx0, rows0, gs0, ss0, is0),
                (sidx1, didx1, rows1, gs1, ss1, is1))

        def _idx_desc(c, k):
            sidx, didx = bufs[k][0], bufs[k][1]
            row = pl.multiple_of((chunk0 + b + c) * _RB, _RB)
            return ((ei3_hbm.at[0].at[pl.ds(row, _RB)], sidx),
                    (ei3_hbm.at[1].at[pl.ds(row, _RB)], didx))

        def load(c, k):
            isem = bufs[k][5]
            for src, dst in _idx_desc(c, k):
                pltpu.async_copy(src, dst, isem)

        def gfire(c, k):
            sidx, didx, rows, gsem, _, isem = bufs[k]
            for src, dst in _idx_desc(c, k):
                pltpu.make_async_copy(src, dst, isem).wait()
            for j in range(_RB):
                pltpu.async_copy(tab.at[sidx.at[j]],
                                 rows.at[pl.ds(j * 128, 128)], gsem)

        def sfire(k):
            sidx, didx, rows, gsem, ssem, _ = bufs[k]
            for j in range(_RB):
                pltpu.make_async_copy(tab.at[sidx.at[j]],
                                      rows.at[pl.ds(j * 128, 128)],
                                      gsem).wait()
            for j in range(_RB):
                pltpu.async_copy(rows.at[pl.ds(j * 128, 128)],
                                 acc.at[didx.at[j]], ssem, add=True)

        def sdrain(k):
            sidx, didx, rows, _, ssem, _2 = bufs[k]
            for j in range(_RB):
                pltpu.make_async_copy(rows.at[pl.ds(j * 128, 128)],
                                      acc.at[didx.at[j]], ssem).wait()

        load(0, 0)
        load(1, 1)
        gfire(0, 0)
        gfire(1, 1)

        def pair(p, carry):
            c0 = 2 * p
            sfire(0)
            sdrain(0)

            @pl.when(c0 + 2 < n)
            def _():
                load(c0 + 2, 0)

            sfire(1)

            @pl.when(c0 + 2 < n)
            def _():
                gfire(c0 + 2, 0)

            sdrain(1)

            @pl.when(c0 + 3 < n)
            def _():
                load(c0 + 3, 1)
                gfire(c0 + 3, 1)

            return carry

        lax.fori_loop(0, n // 2, pair, 0)

        @pl.when(n % 2 == 1)
        def _():
            sfire(0)
            sdrain(0)

        plsc.subcore_barrier()

        @pl.when(cid == 0)
        def _():
            _node_copy(sid, lambda b2, n2: acc.at[pl.ds(b2, n2)],
                       lambda b2, n2: out0_hbm.at[pl.ds(b2, n2)])

        @pl.when(cid == 1)
        def _():
            _node_copy(sid, lambda b2, n2: acc.at[pl.ds(b2, n2)],
                       lambda b2, n2: out1_hbm.at[pl.ds(b2, n2)])

    return _agg


_agg1_kernel = _make_agg_kernel(True)
_agg2_kernel = _make_agg_kernel(False)


def _gelu(x):
    return 0.5 * x * (1.0 + lax.erf(x * 0.7071067811865476))


def _ln(x, g, b, eps=1e-5):
    m = jnp.mean(x, axis=-1, keepdims=True)
    v = jnp.mean((x - m) ** 2, axis=-1, keepdims=True)
    return (x - m) / jnp.sqrt(v + eps) * g + b


_RN = 1984
_RP = _RN // 8
_GRID = (_N + _RN - 1) // _RN


def _pack16(z, scale8=None):
    z3 = z.reshape(_RP, 8, _O)
    parts = []
    for s in range(8):
        p = z3[:, s, :]
        if scale8 is not None:
            p = p * scale8[:, s:s + 1]
        parts.append(p)
    return jnp.concatenate(parts, axis=1)


def _unpack16(zp):
    parts = [zp[:, 16 * s:16 * (s + 1)].reshape(_RP, 1, _O) for s in range(8)]
    return jnp.concatenate(parts, axis=1).reshape(_RN, _O)


def _full(shape):
    return pl.BlockSpec(shape, lambda i: tuple(0 for _ in shape))


def _enc_body(x0, x1, d0, d1, W10, b10, g10, bb10, W11, b11, g11, bb11,
              Wb0, b2t0, rg0, rb0, Wb1, b2t1, rg1, rb1, dsel, kJ,
              zcp, dinv8):
    di8 = lax.rsqrt(d0[...] + d1[...] + 1.0)
    dscale = jnp.dot(di8, dsel[...], preferred_element_type=jnp.float32)

    def hidden(x, W1, b1, g1, bb1):
        h = jnp.dot(x, W1, preferred_element_type=jnp.float32) + b1
        return _gelu(_ln(h, g1, bb1))

    h0 = hidden(x0[...], W10[...], b10[...], g10[...], bb10[...])
    h1 = hidden(x1[...], W11[...], b11[...], g11[...], bb11[...])
    h3 = jnp.concatenate([h0, h1], axis=1).reshape(_RP, 8, _D)
    hP = jnp.concatenate([h3[:, s, :] for s in range(8)], axis=1)

    def head(Wb, b2t, rg, rb):
        h2 = jnp.dot(hP, Wb[...], preferred_element_type=jnp.float32) + b2t[...]
        t = h2 + h2
        m = jnp.dot(t, kJ[...], preferred_element_type=jnp.float32)
        c = t - m
        v = jnp.dot(c * c, kJ[...], preferred_element_type=jnp.float32)
        return (c * lax.rsqrt(v + 1e-5) * rg[...] + rb[...]) * dscale

    zcp[0] = head(Wb0, b2t0, rg0, rb0)
    zcp[1] = head(Wb1, b2t1, rg1, rb1)
    dinv8[...] = di8


def _enc_call(x0, x1, d0, d1, w):
    grid = _GRID
    in_specs = [
        pl.BlockSpec((_RN, _D), lambda i: (i, 0)),
        pl.BlockSpec((_RN, _D), lambda i: (i, 0)),
        pl.BlockSpec((_RP, 8), lambda i: (i, 0)),
        pl.BlockSpec((_RP, 8), lambda i: (i, 0)),
    ] + [_full(a.shape) for a in w]
    return pl.pallas_call(
        _enc_body,
        grid=(grid,),
        in_specs=in_specs,
        out_specs=[pl.BlockSpec((_NC, _RP, 128), lambda i: (0, i, 0)),
                   pl.BlockSpec((_RP, 8), lambda i: (i, 0))],
        out_shape=[jax.ShapeDtypeStruct((_NC, _P, 128), jnp.float32),
                   jax.ShapeDtypeStruct((_P, 8), jnp.float32)],
    )(x0, x1, d0, d1, *w)


def _mid_body(S0p, S1p, dinv8, dsel, combk0, combk1, combb, kJ, g128, bb128,
              clfW1k, clfb1, clfW2k, clfb2, zp_pk, zpp_out, domp_out):
    dscale = jnp.dot(dinv8[...], dsel[...],
                     preferred_element_type=jnp.float32)
    t = (jnp.dot(S0p[...] * dscale, combk0[...],
                 preferred_element_type=jnp.float32)
         + jnp.dot(S1p[...] * dscale, combk1[...],
                   preferred_element_type=jnp.float32) + combb[...])
    m = jnp.dot(t, kJ[...], preferred_element_type=jnp.float32)
    c = t - m
    v = jnp.dot(c * c, kJ[...], preferred_element_type=jnp.float32)
    zP = c * lax.rsqrt(v + 1e-5) * g128[...] + bb128[...]
    h = _gelu(jnp.dot(zP, clfW1k[...], preferred_element_type=jnp.float32)
              + clfb1[...])
    domP = jnp.dot(h, clfW2k[...], preferred_element_type=jnp.float32) \
        + clfb2[...]
    zp_pk[...] = zP
    zpp_out[...] = zP * dscale
    domp_out[...] = domP


def _mid_call(S0p, S1p, dinv8, w):
    grid = _GRID
    in_specs = [
        pl.BlockSpec((_RP, 128), lambda i: (i, 0)),
        pl.BlockSpec((_RP, 128), lambda i: (i, 0)),
        pl.BlockSpec((_RP, 8), lambda i: (i, 0)),
    ] + [_full(a.shape) for a in w]
    return pl.pallas_call(
        _mid_body,
        grid=(grid,),
        in_specs=in_specs,
        out_specs=[pl.BlockSpec((_RP, 128), lambda i: (i, 0)),
                   pl.BlockSpec((_RP, 128), lambda i: (i, 0)),
                   pl.BlockSpec((_RP, 64), lambda i: (i, 0))],
        out_shape=[jax.ShapeDtypeStruct((_P, 128), jnp.float32),
                   jax.ShapeDtypeStruct((_P, 128), jnp.float32),
                   jax.ShapeDtypeStruct((_P, 64), jnp.float32)],
    )(S0p, S1p, dinv8, *w)


def _fin_body(S2a, S2b, zpp, dinv8, dsel, W0, b0, W1, b1, rec0, rec1):
    dscale = jnp.dot(dinv8[...], dsel[...],
                     preferred_element_type=jnp.float32)
    sum_p = (S2a[...] + S2b[...] - zpp[...]) * dscale
    aggz = _unpack16(sum_p)
    rec0[...] = jnp.dot(aggz, W0[...], preferred_element_type=jnp.float32) + b0[...]
    rec1[...] = jnp.dot(aggz, W1[...], preferred_element_type=jnp.float32) + b1[...]


def _fin_call(S2a, S2b, zpp, dinv8, w):
    grid = _GRID
    in_specs = [
        pl.BlockSpec((_RP, 128), lambda i: (i, 0)),
        pl.BlockSpec((_RP, 128), lambda i: (i, 0)),
        pl.BlockSpec((_RP, 128), lambda i: (i, 0)),
        pl.BlockSpec((_RP, 8), lambda i: (i, 0)),
    ] + [_full(a.shape) for a in w]
    return pl.pallas_call(
        _fin_body,
        grid=(grid,),
        in_specs=in_specs,
        out_specs=[pl.BlockSpec((_RN, _D), lambda i: (i, 0)),
                   pl.BlockSpec((_RN, _D), lambda i: (i, 0))],
        out_shape=[jax.ShapeDtypeStruct((_N, _D), jnp.float32),
                   jax.ShapeDtypeStruct((_N, _D), jnp.float32)],
    )(S2a, S2b, zpp, dinv8, *w)


def kernel(x0, x1, edge_index, enc0_W1, enc0_b1, enc0_g1, enc0_bb1, enc0_W2,
           enc0_b2, enc0_rg, enc0_rb, enc1_W1, enc1_b1, enc1_g1, enc1_bb1,
           enc1_W2, enc1_b2, enc1_rg, enc1_rb, comb_W, comb_b, comb_g,
           comb_bb, dec0_W, dec0_b, dec1_W, dec1_b, clf_W1, clf_b1, clf_W2,
           clf_b2):
    f32 = jnp.float32
    ei3 = edge_index.reshape(2, _ER, 128)
    zcol = jnp.zeros((_N,), f32)
    ones128 = jnp.ones((128,), f32)

    deg0, deg1 = _deg_kernel(ei3, zcol, ones128)

    row = lambda a: a.reshape(1, -1)
    eye8 = jnp.eye(8, dtype=f32)
    kron8 = lambda W: jnp.kron(eye8, W)
    tile8 = lambda v: jnp.tile(v, 8).reshape(1, -1)
    dsel = kron8(jnp.ones((1, _O), f32))
    kJ = kron8(jnp.full((_O, _O), 1.0 / _O, f32))
    z64 = jnp.zeros((_H, _O), f32)
    enc_w = (enc0_W1, row(enc0_b1), row(enc0_g1), row(enc0_bb1),
             enc1_W1, row(enc1_b1), row(enc1_g1), row(enc1_bb1),
             kron8(jnp.concatenate([enc0_W2, z64], axis=0)), tile8(enc0_b2),
             tile8(enc0_rg), tile8(enc0_rb),
             kron8(jnp.concatenate([z64, enc1_W2], axis=0)), tile8(enc1_b2),
             tile8(enc1_rg), tile8(enc1_rb), dsel, kJ)
    zcp_p, dinv8 = _enc_call(x0, x1, deg0.reshape(_P, 8), deg1.reshape(_P, 8),
                             enc_w)

    S0, S1 = _agg1_kernel(zcp_p.reshape(_NC, _N, _O), ei3)

    mid_w = (dsel, kron8(comb_W[:_O]), kron8(comb_W[_O:]), tile8(comb_b),
             kJ, tile8(comb_g), tile8(comb_bb), kron8(clf_W1), tile8(clf_b1),
             kron8(clf_W2), tile8(clf_b2))
    zP, zpP, domP = _mid_call(S0.reshape(_P, 128), S1.reshape(_P, 128),
                              dinv8, mid_w)

    S2a, S2b = _agg2_kernel(zpP.reshape(_N, _O), ei3)

    rec0, rec1 = _fin_call(S2a.reshape(_P, 128), S2b.reshape(_P, 128),
                           zpP, dinv8,
                           (dsel, dec0_W, row(dec0_b), dec1_W, row(dec1_b)))
    return (zP.reshape(_N, _O), rec0, rec1, domP.reshape(_N, 8))


# speedup vs baseline: 49.9641x; 1.0185x over previous
import functools

import jax
import jax.numpy as jnp
from jax import lax
from jax.experimental import pallas as pl
from jax.experimental.pallas import tpu as pltpu
from jax.experimental.pallas import tpu_sc as plsc

_N = 100000
_E = 1600000
_D = 128
_H = 64
_O = 16
_NC = 2
_NS = 16
_P = _N // 8
_ER = _E // 128
_RB = 4
_NCH = _ER // _RB
_C0_CH = 1562

_mesh = plsc.VectorSubcoreMesh(core_axis_name="c", subcore_axis_name="s",
                               num_cores=_NC, num_subcores=_NS)
_sc_params = pltpu.CompilerParams(use_tc_tiling_on_sc=False)

_OCT_Q, _OCT_R = divmod(_N // 8, _NS)


def _node_copy(sid, src_at, dst_at):
    base = pl.multiple_of((sid * _OCT_Q + jnp.minimum(sid, _OCT_R)) * 8, 8)
    pltpu.sync_copy(src_at(base, _OCT_Q * 8), dst_at(base, _OCT_Q * 8))

    @pl.when(sid < _OCT_R)
    def _():
        b1 = pl.multiple_of(base + _OCT_Q * 8, 8)
        pltpu.sync_copy(src_at(b1, 8), dst_at(b1, 8))


def _chunk_sched(nchunks, rm, sid):
    q = nchunks // _NS
    b = sid * q + jnp.minimum(sid, rm)
    n = q + jnp.where(sid < rm, 1, 0)
    return b, n


@functools.partial(
    pl.kernel,
    out_type=[jax.ShapeDtypeStruct((_N,), jnp.float32),
              jax.ShapeDtypeStruct((_N,), jnp.float32)],
    mesh=_mesh,
    compiler_params=_sc_params,
    scratch_types=[
        pltpu.VMEM((_RB, 128), jnp.int32),
        pltpu.VMEM((_RB, 128), jnp.int32),
        pltpu.VMEM((128,), jnp.float32),
        pltpu.VMEM_SHARED((_N,), jnp.float32),
        pltpu.SemaphoreType.DMA,
        pltpu.SemaphoreType.DMA,
        pltpu.SemaphoreType.DMA,
        pltpu.SemaphoreType.DMA,
    ],
)
def _deg_kernel(ei3_hbm, zcol_hbm, ones_hbm, out0_hbm, out1_hbm,
                didx0, didx1, ones_v, acc, sem0, sem1, is0, is1):
    cid = lax.axis_index("c")
    sid = lax.axis_index("s")
    pltpu.sync_copy(ones_hbm, ones_v)
    _node_copy(sid, lambda b, n: zcol_hbm.at[pl.ds(b, n)],
               lambda b, n: acc.at[pl.ds(b, n)])
    plsc.subcore_barrier()

    chunk0 = cid * _C0_CH
    b, n = _chunk_sched(_C0_CH, jnp.where(cid == 0, _C0_CH % _NS,
                                          (_NCH - _C0_CH) % _NS), sid)
    bufs = ((didx0, sem0, is0), (didx1, sem1, is1))

    def _isrc(c):
        row = pl.multiple_of((chunk0 + b + c) * _RB, _RB)
        return ei3_hbm.at[1].at[pl.ds(row, _RB)]

    def load(c, k):
        pltpu.async_copy(_isrc(c), bufs[k][0], bufs[k][2])

    def sfire(c, k):
        didx, sem, isem = bufs[k]
        pltpu.make_async_copy(_isrc(c), didx, isem).wait()
        for j in range(_RB):
            pltpu.async_copy(ones_v, acc.at[didx.at[j]], sem, add=True)

    def sdrain(k):
        didx, sem, _ = bufs[k]
        for j in range(_RB):
            pltpu.make_async_copy(ones_v, acc.at[didx.at[j]], sem).wait()

    load(0, 0)
    load(1, 1)

    def pair(p, carry):
        c0 = 2 * p
        sfire(c0, 0)
        sdrain(0)

        @pl.when(c0 + 2 < n)
        def _():
            load(c0 + 2, 0)

        sfire(c0 + 1, 1)
        sdrain(1)

        @pl.when(c0 + 3 < n)
        def _():
            load(c0 + 3, 1)

        return carry

    lax.fori_loop(0, n // 2, pair, 0)

    @pl.when(n % 2 == 1)
    def _():
        sfire(n - 1, 0)
        sdrain(0)

    plsc.subcore_barrier()

    @pl.when(cid == 0)
    def _():
        _node_copy(sid, lambda b2, n2: acc.at[pl.ds(b2, n2)],
                   lambda b2, n2: out0_hbm.at[pl.ds(b2, n2)])

    @pl.when(cid == 1)
    def _():
        _node_copy(sid, lambda b2, n2: acc.at[pl.ds(b2, n2)],
                   lambda b2, n2: out1_hbm.at[pl.ds(b2, n2)])


def _make_agg_kernel(per_core_features):
    @functools.partial(
        pl.kernel,
        out_type=[jax.ShapeDtypeStruct((_N, _O), jnp.float32),
                  jax.ShapeDtypeStruct((_N, _O), jnp.float32)],
        mesh=_mesh,
        compiler_params=_sc_params,
        scratch_types=[
            pltpu.VMEM((_RB, 128), jnp.int32),
            pltpu.VMEM((_RB, 128), jnp.int32),
            pltpu.VMEM((_RB, 128), jnp.int32),
            pltpu.VMEM((_RB, 128), jnp.int32),
            pltpu.VMEM((_RB * 128, _O), jnp.float32),
            pltpu.VMEM((_RB * 128, _O), jnp.float32),
            pltpu.VMEM_SHARED((_N, _O), jnp.float32),
            pltpu.SemaphoreType.DMA,
            pltpu.SemaphoreType.DMA,
            pltpu.SemaphoreType.DMA,
            pltpu.SemaphoreType.DMA,
            pltpu.SemaphoreType.DMA,
            pltpu.SemaphoreType.DMA,
        ],
    )
    def _agg(tab_hbm, ei3_hbm, out0_hbm, out1_hbm, sidx0, didx0, sidx1, didx1,
             rows0, rows1, acc, gs0, ss0, gs1, ss1, is0, is1):
        cid = lax.axis_index("c")
        sid = lax.axis_index("s")
        tab = tab_hbm.at[cid] if per_core_features else tab_hbm
        _node_copy(sid, lambda b, n: tab.at[pl.ds(b, n)],
                   lambda b, n: acc.at[pl.ds(b, n)])
        plsc.subcore_barrier()

        if per_core_features:
            chunk0 = 0
            b, n = _chunk_sched(_NCH, _NCH % _NS, sid)
        else:
            chunk0 = cid * _C0_CH
            b, n = _chunk_sched(_C0_CH, jnp.where(cid == 0, _C0_CH % _NS,
                                                  (_NCH - _C0_CH) % _NS), sid)

        bufs = ((sidx0, didx0, rows0, gs0, ss0, is0),
                (sidx1, didx1, rows1, gs1, ss1, is1))

        def _idx_desc(c, k):
            sidx, didx = bufs[k][0], bufs[k][1]
            row = pl.multiple_of((chunk0 + b + c) * _RB, _RB)
            return ((ei3_hbm.at[0].at[pl.ds(row, _RB)], sidx),
                    (ei3_hbm.at[1].at[pl.ds(row, _RB)], didx))

        def load(c, k):
            isem = bufs[k][5]
            for src, dst in _idx_desc(c, k):
                pltpu.async_copy(src, dst, isem)

        def gfire(c, k):
            sidx, didx, rows, gsem, _, isem = bufs[k]
            for src, dst in _idx_desc(c, k):
                pltpu.make_async_copy(src, dst, isem).wait()
            for j in range(_RB):
                pltpu.async_copy(tab.at[sidx.at[j]],
                                 rows.at[pl.ds(j * 128, 128)], gsem)

        def sfire(k):
            sidx, didx, rows, gsem, ssem, _ = bufs[k]
            for j in range(_RB):
                pltpu.make_async_copy(tab.at[sidx.at[j]],
                                      rows.at[pl.ds(j * 128, 128)],
                                      gsem).wait()
            for j in range(_RB):
                pltpu.async_copy(rows.at[pl.ds(j * 128, 128)],
                                 acc.at[didx.at[j]], ssem, add=True)

        def sdrain(k):
            sidx, didx, rows, _, ssem, _2 = bufs[k]
            for j in range(_RB):
                pltpu.make_async_copy(rows.at[pl.ds(j * 128, 128)],
                                      acc.at[didx.at[j]], ssem).wait()

        load(0, 0)
        load(1, 1)
        gfire(0, 0)
        gfire(1, 1)

        def pair(p, carry):
            c0 = 2 * p
            sfire(0)
            sdrain(0)

            @pl.when(c0 + 2 < n)
            def _():
                load(c0 + 2, 0)

            sfire(1)

            @pl.when(c0 + 2 < n)
            def _():
                gfire(c0 + 2, 0)

            sdrain(1)

            @pl.when(c0 + 3 < n)
            def _():
                load(c0 + 3, 1)
                gfire(c0 + 3, 1)

            return carry

        lax.fori_loop(0, n // 2, pair, 0)

        @pl.when(n % 2 == 1)
        def _():
            sfire(0)
            sdrain(0)

        plsc.subcore_barrier()

        @pl.when(cid == 0)
        def _():
            _node_copy(sid, lambda b2, n2: acc.at[pl.ds(b2, n2)],
                       lambda b2, n2: out0_hbm.at[pl.ds(b2, n2)])

        @pl.when(cid == 1)
        def _():
            _node_copy(sid, lambda b2, n2: acc.at[pl.ds(b2, n2)],
                       lambda b2, n2: out1_hbm.at[pl.ds(b2, n2)])

    return _agg


_agg1_kernel = _make_agg_kernel(True)
_agg2_kernel = _make_agg_kernel(False)


def _gelu(x):
    return 0.5 * x * (1.0 + lax.erf(x * 0.7071067811865476))


def _ln(x, g, b, eps=1e-5):
    m = jnp.mean(x, axis=-1, keepdims=True)
    v = jnp.mean((x - m) ** 2, axis=-1, keepdims=True)
    return (x - m) / jnp.sqrt(v + eps) * g + b


_RN = 1984
_RP = _RN // 8
_GRID = (_N + _RN - 1) // _RN


def _pack16(z, scale8=None):
    z3 = z.reshape(_RP, 8, _O)
    parts = []
    for s in range(8):
        p = z3[:, s, :]
        if scale8 is not None:
            p = p * scale8[:, s:s + 1]
        parts.append(p)
    return jnp.concatenate(parts, axis=1)


def _unpack16(zp):
    parts = [zp[:, 16 * s:16 * (s + 1)].reshape(_RP, 1, _O) for s in range(8)]
    return jnp.concatenate(parts, axis=1).reshape(_RN, _O)


def _full(shape):
    return pl.BlockSpec(shape, lambda i: tuple(0 for _ in shape))


def _enc_body(x0, x1, d0, d1, W10, b10, g10, bb10, W11, b11, g11, bb11,
              Wb0, b2t0, rg0, rb0, Wb1, b2t1, rg1, rb1, dsel, kJ,
              zcp, dinv8):
    di8 = lax.rsqrt(d0[...] + d1[...] + 1.0)
    dscale = jnp.dot(di8, dsel[...], preferred_element_type=jnp.float32)

    def hidden(x, W1, b1, g1, bb1):
        h = jnp.dot(x, W1, preferred_element_type=jnp.float32) + b1
        return _gelu(_ln(h, g1, bb1))

    h0 = hidden(x0[...], W10[...], b10[...], g10[...], bb10[...])
    h1 = hidden(x1[...], W11[...], b11[...], g11[...], bb11[...])
    h3 = jnp.concatenate([h0, h1], axis=1).reshape(_RP, 8, _D)
    hP = jnp.concatenate([h3[:, s, :] for s in range(8)], axis=1)

    def head(Wb, b2t, rg, rb):
        h2 = jnp.dot(hP, Wb[...], preferred_element_type=jnp.float32) + b2t[...]
        t = h2 + h2
        m = jnp.dot(t, kJ[...], preferred_element_type=jnp.float32)
        c = t - m
        v = jnp.dot(c * c, kJ[...], preferred_element_type=jnp.float32)
        return (c * lax.rsqrt(v + 1e-5) * rg[...] + rb[...]) * dscale

    zcp[0] = head(Wb0, b2t0, rg0, rb0)
    zcp[1] = head(Wb1, b2t1, rg1, rb1)
    dinv8[...] = di8


def _enc_call(x0, x1, d0, d1, w):
    grid = _GRID
    in_specs = [
        pl.BlockSpec((_RN, _D), lambda i: (i, 0)),
        pl.BlockSpec((_RN, _D), lambda i: (i, 0)),
        pl.BlockSpec((_RP, 8), lambda i: (i, 0)),
        pl.BlockSpec((_RP, 8), lambda i: (i, 0)),
    ] + [_full(a.shape) for a in w]
    return pl.pallas_call(
        _enc_body,
        grid=(grid,),
        in_specs=in_specs,
        out_specs=[pl.BlockSpec((_NC, _RP, 128), lambda i: (0, i, 0)),
                   pl.BlockSpec((_RP, 8), lambda i: (i, 0))],
        out_shape=[jax.ShapeDtypeStruct((_NC, _P, 128), jnp.float32),
                   jax.ShapeDtypeStruct((_P, 8), jnp.float32)],
    )(x0, x1, d0, d1, *w)


def _mid_body(S0p, S1p, dinv8, dsel, combk0, combk1, combb, kJ, g128, bb128,
              clfW1k, clfb1, clfW2k, clfb2, zp_pk, zpp_out, domp_out):
    dscale = jnp.dot(dinv8[...], dsel[...],
                     preferred_element_type=jnp.float32)
    t = (jnp.dot(S0p[...] * dscale, combk0[...],
                 preferred_element_type=jnp.float32)
         + jnp.dot(S1p[...] * dscale, combk1[...],
                   preferred_element_type=jnp.float32) + combb[...])
    m = jnp.dot(t, kJ[...], preferred_element_type=jnp.float32)
    c = t - m
    v = jnp.dot(c * c, kJ[...], preferred_element_type=jnp.float32)
    zP = c * lax.rsqrt(v + 1e-5) * g128[...] + bb128[...]
    h = _gelu(jnp.dot(zP, clfW1k[...], preferred_element_type=jnp.float32)
              + clfb1[...])
    domP = jnp.dot(h, clfW2k[...], preferred_element_type=jnp.float32) \
        + clfb2[...]
    zp_pk[...] = zP
    zpp_out[...] = zP * dscale
    domp_out[...] = domP


def _mid_call(S0p, S1p, dinv8, w):
    grid = _GRID
    in_specs = [
        pl.BlockSpec((_RP, 128), lambda i: (i, 0)),
        pl.BlockSpec((_RP, 128), lambda i: (i, 0)),
        pl.BlockSpec((_RP, 8), lambda i: (i, 0)),
    ] + [_full(a.shape) for a in w]
    return pl.pallas_call(
        _mid_body,
        grid=(grid,),
        in_specs=in_specs,
        out_specs=[pl.BlockSpec((_RP, 128), lambda i: (i, 0)),
                   pl.BlockSpec((_RP, 128), lambda i: (i, 0)),
                   pl.BlockSpec((_RP, 64), lambda i: (i, 0))],
        out_shape=[jax.ShapeDtypeStruct((_P, 128), jnp.float32),
                   jax.ShapeDtypeStruct((_P, 128), jnp.float32),
                   jax.ShapeDtypeStruct((_P, 64), jnp.float32)],
    )(S0p, S1p, dinv8, *w)


def _fin_body(S2a, S2b, zpp, dinv8, dsel, W0, b0, W1, b1, rec0, rec1):
    dscale = jnp.dot(dinv8[...], dsel[...],
                     preferred_element_type=jnp.float32)
    sum_p = (S2a[...] + S2b[...] - zpp[...]) * dscale
    aggz = _unpack16(sum_p)
    rec0[...] = jnp.dot(aggz, W0[...], preferred_element_type=jnp.float32) + b0[...]
    rec1[...] = jnp.dot(aggz, W1[...], preferred_element_type=jnp.float32) + b1[...]


def _fin_call(S2a, S2b, zpp, dinv8, w):
    grid = _GRID
    in_specs = [
        pl.BlockSpec((_RP, 128), lambda i: (i, 0)),
        pl.BlockSpec((_RP, 128), lambda i: (i, 0)),
        pl.BlockSpec((_RP, 128), lambda i: (i, 0)),
        pl.BlockSpec((_RP, 8), lambda i: (i, 0)),
    ] + [_full(a.shape) for a in w]
    return pl.pallas_call(
        _fin_body,
        grid=(grid,),
        in_specs=in_specs,
        out_specs=[pl.BlockSpec((_RN, _D), lambda i: (i, 0)),
                   pl.BlockSpec((_RN, _D), lambda i: (i, 0))],
        out_shape=[jax.ShapeDtypeStruct((_N, _D), jnp.float32),
                   jax.ShapeDtypeStruct((_N, _D), jnp.float32)],
    )(S2a, S2b, zpp, dinv8, *w)


def kernel(x0, x1, edge_index, enc0_W1, enc0_b1, enc0_g1, enc0_bb1, enc0_W2,
           enc0_b2, enc0_rg, enc0_rb, enc1_W1, enc1_b1, enc1_g1, enc1_bb1,
           enc1_W2, enc1_b2, enc1_rg, enc1_rb, comb_W, comb_b, comb_g,
           comb_bb, dec0_W, dec0_b, dec1_W, dec1_b, clf_W1, clf_b1, clf_W2,
           clf_b2):
    f32 = jnp.float32
    ei3 = edge_index.reshape(2, _ER, 128)
    zcol = jnp.zeros((_N,), f32)
    ones128 = jnp.ones((128,), f32)

    deg0, deg1 = _deg_kernel(ei3, zcol, ones128)

    row = lambda a: a.reshape(1, -1)
    eye8 = jnp.eye(8, dtype=f32)
    kron8 = lambda W: jnp.kron(eye8, W)
    tile8 = lambda v: jnp.tile(v, 8).reshape(1, -1)
    dsel = kron8(jnp.ones((1, _O), f32))
    kJ = kron8(jnp.full((_O, _O), 1.0 / _O, f32))
    z64 = jnp.zeros((_H, _O), f32)
    enc_w = (enc0_W1, row(enc0_b1), row(enc0_g1), row(enc0_bb1),
             enc1_W1, row(enc1_b1), row(enc1_g1), row(enc1_bb1),
             kron8(jnp.concatenate([enc0_W2, z64], axis=0)), tile8(enc0_b2),
             tile8(enc0_rg), tile8(enc0_rb),
             kron8(jnp.concatenate([z64, enc1_W2], axis=0)), tile8(enc1_b2),
             tile8(enc1_rg), tile8(enc1_rb), dsel, kJ)
    zcp_p, dinv8 = _enc_call(x0, x1, deg0.reshape(_P, 8), deg1.reshape(_P, 8),
                             enc_w)

    S0, S1 = _agg1_kernel(zcp_p.reshape(_NC, _N, _O), ei3)

    mid_w = (dsel, kron8(comb_W[:_O]), kron8(comb_W[_O:]), tile8(comb_b),
             kJ, tile8(comb_g), tile8(comb_bb), kron8(clf_W1), tile8(clf_b1),
             kron8(clf_W2), tile8(clf_b2))
    zP, zpP, domP = _mid_call(S0.reshape(_P, 128), S1.reshape(_P, 128),
                              dinv8, mid_w)

    S2a, S2b = _agg2_kernel(zpP.reshape(_N, _O), ei3)

    rec0, rec1 = _fin_call(S2a.reshape(_P, 128), S2b.reshape(_P, 128),
                           zpP, dinv8,
                           (dsel, dec0_W, row(dec0_b), dec1_W, row(dec1_b)))
    return (zP.reshape(_N, _O), rec0, rec1, domP.reshape(_N, 8))


# speedup vs baseline: 52.3496x; 1.0477x over previous
import functools

import jax
import jax.numpy as jnp
from jax import lax
from jax.experimental import pallas as pl
from jax.experimental.pallas import tpu as pltpu
from jax.experimental.pallas import tpu_sc as plsc

_N = 100000
_E = 1600000
_D = 128
_H = 64
_O = 16
_NC = 2
_NS = 16
_P = _N // 8
_ER = _E // 128
_RB = 5
_NCH = _ER // _RB
_C0_CH = _NCH // 2

_mesh = plsc.VectorSubcoreMesh(core_axis_name="c", subcore_axis_name="s",
                               num_cores=_NC, num_subcores=_NS)
_sc_params = pltpu.CompilerParams(use_tc_tiling_on_sc=False)

_OCT_Q, _OCT_R = divmod(_N // 8, _NS)


def _node_copy(sid, src_at, dst_at):
    base = pl.multiple_of((sid * _OCT_Q + jnp.minimum(sid, _OCT_R)) * 8, 8)
    pltpu.sync_copy(src_at(base, _OCT_Q * 8), dst_at(base, _OCT_Q * 8))

    @pl.when(sid < _OCT_R)
    def _():
        b1 = pl.multiple_of(base + _OCT_Q * 8, 8)
        pltpu.sync_copy(src_at(b1, 8), dst_at(b1, 8))


def _chunk_sched(nchunks, rm, sid):
    q = nchunks // _NS
    b = sid * q + jnp.minimum(sid, rm)
    n = q + jnp.where(sid < rm, 1, 0)
    return b, n


@functools.partial(
    pl.kernel,
    out_type=[jax.ShapeDtypeStruct((_N,), jnp.float32),
              jax.ShapeDtypeStruct((_N,), jnp.float32)],
    mesh=_mesh,
    compiler_params=_sc_params,
    scratch_types=[
        pltpu.VMEM((_RB, 128), jnp.int32),
        pltpu.VMEM((_RB, 128), jnp.int32),
        pltpu.VMEM((128,), jnp.float32),
        pltpu.VMEM_SHARED((_N,), jnp.float32),
        pltpu.SemaphoreType.DMA,
        pltpu.SemaphoreType.DMA,
        pltpu.SemaphoreType.DMA,
        pltpu.SemaphoreType.DMA,
    ],
)
def _deg_kernel(ei3_hbm, zcol_hbm, ones_hbm, out0_hbm, out1_hbm,
                didx0, didx1, ones_v, acc, sem0, sem1, is0, is1):
    cid = lax.axis_index("c")
    sid = lax.axis_index("s")
    pltpu.sync_copy(ones_hbm, ones_v)
    _node_copy(sid, lambda b, n: zcol_hbm.at[pl.ds(b, n)],
               lambda b, n: acc.at[pl.ds(b, n)])
    plsc.subcore_barrier()

    chunk0 = cid * _C0_CH
    b, n = _chunk_sched(_C0_CH, jnp.where(cid == 0, _C0_CH % _NS,
                                          (_NCH - _C0_CH) % _NS), sid)
    bufs = ((didx0, sem0, is0), (didx1, sem1, is1))

    def _isrc(c):
        row = pl.multiple_of((chunk0 + b + c) * _RB, _RB)
        return ei3_hbm.at[1].at[pl.ds(row, _RB)]

    def load(c, k):
        pltpu.async_copy(_isrc(c), bufs[k][0], bufs[k][2])

    def sfire(c, k):
        didx, sem, isem = bufs[k]
        pltpu.make_async_copy(_isrc(c), didx, isem).wait()
        for j in range(_RB):
            pltpu.async_copy(ones_v, acc.at[didx.at[j]], sem, add=True)

    def sdrain(k):
        didx, sem, _ = bufs[k]
        for j in range(_RB):
            pltpu.make_async_copy(ones_v, acc.at[didx.at[j]], sem).wait()

    load(0, 0)
    load(1, 1)

    def pair(p, carry):
        c0 = 2 * p
        sfire(c0, 0)
        sdrain(0)

        @pl.when(c0 + 2 < n)
        def _():
            load(c0 + 2, 0)

        sfire(c0 + 1, 1)
        sdrain(1)

        @pl.when(c0 + 3 < n)
        def _():
            load(c0 + 3, 1)

        return carry

    lax.fori_loop(0, n // 2, pair, 0)

    @pl.when(n % 2 == 1)
    def _():
        sfire(n - 1, 0)
        sdrain(0)

    plsc.subcore_barrier()

    @pl.when(cid == 0)
    def _():
        _node_copy(sid, lambda b2, n2: acc.at[pl.ds(b2, n2)],
                   lambda b2, n2: out0_hbm.at[pl.ds(b2, n2)])

    @pl.when(cid == 1)
    def _():
        _node_copy(sid, lambda b2, n2: acc.at[pl.ds(b2, n2)],
                   lambda b2, n2: out1_hbm.at[pl.ds(b2, n2)])


def _make_agg_kernel(per_core_features):
    @functools.partial(
        pl.kernel,
        out_type=[jax.ShapeDtypeStruct((_N, _O), jnp.float32),
                  jax.ShapeDtypeStruct((_N, _O), jnp.float32)],
        mesh=_mesh,
        compiler_params=_sc_params,
        scratch_types=[
            pltpu.VMEM((_RB, 128), jnp.int32),
            pltpu.VMEM((_RB, 128), jnp.int32),
            pltpu.VMEM((_RB, 128), jnp.int32),
            pltpu.VMEM((_RB, 128), jnp.int32),
            pltpu.VMEM((_RB * 128, _O), jnp.float32),
            pltpu.VMEM((_RB * 128, _O), jnp.float32),
            pltpu.VMEM_SHARED((_N, _O), jnp.float32),
            pltpu.SemaphoreType.DMA,
            pltpu.SemaphoreType.DMA,
            pltpu.SemaphoreType.DMA,
            pltpu.SemaphoreType.DMA,
            pltpu.SemaphoreType.DMA,
            pltpu.SemaphoreType.DMA,
        ],
    )
    def _agg(tab_hbm, ei3_hbm, out0_hbm, out1_hbm, sidx0, didx0, sidx1, didx1,
             rows0, rows1, acc, gs0, ss0, gs1, ss1, is0, is1):
        cid = lax.axis_index("c")
        sid = lax.axis_index("s")
        tab = tab_hbm.at[cid] if per_core_features else tab_hbm
        _node_copy(sid, lambda b, n: tab.at[pl.ds(b, n)],
                   lambda b, n: acc.at[pl.ds(b, n)])
        plsc.subcore_barrier()

        if per_core_features:
            chunk0 = 0
            b, n = _chunk_sched(_NCH, _NCH % _NS, sid)
        else:
            chunk0 = cid * _C0_CH
            b, n = _chunk_sched(_C0_CH, jnp.where(cid == 0, _C0_CH % _NS,
                                                  (_NCH - _C0_CH) % _NS), sid)

        bufs = ((sidx0, didx0, rows0, gs0, ss0, is0),
                (sidx1, didx1, rows1, gs1, ss1, is1))

        def _idx_desc(c, k):
            sidx, didx = bufs[k][0], bufs[k][1]
            row = pl.multiple_of((chunk0 + b + c) * _RB, _RB)
            return ((ei3_hbm.at[0].at[pl.ds(row, _RB)], sidx),
                    (ei3_hbm.at[1].at[pl.ds(row, _RB)], didx))

        def load(c, k):
            isem = bufs[k][5]
            for src, dst in _idx_desc(c, k):
                pltpu.async_copy(src, dst, isem)

        def gfire(c, k):
            sidx, didx, rows, gsem, _, isem = bufs[k]
            for src, dst in _idx_desc(c, k):
                pltpu.make_async_copy(src, dst, isem).wait()
            for j in range(_RB):
                pltpu.async_copy(tab.at[sidx.at[j]],
                                 rows.at[pl.ds(j * 128, 128)], gsem)

        def sfire(k):
            sidx, didx, rows, gsem, ssem, _ = bufs[k]
            for j in range(_RB):
                pltpu.make_async_copy(tab.at[sidx.at[j]],
                                      rows.at[pl.ds(j * 128, 128)],
                                      gsem).wait()
            for j in range(_RB):
                pltpu.async_copy(rows.at[pl.ds(j * 128, 128)],
                                 acc.at[didx.at[j]], ssem, add=True)

        def sdrain(k):
            sidx, didx, rows, _, ssem, _2 = bufs[k]
            for j in range(_RB):
                pltpu.make_async_copy(rows.at[pl.ds(j * 128, 128)],
                                      acc.at[didx.at[j]], ssem).wait()

        load(0, 0)
        load(1, 1)
        gfire(0, 0)
        gfire(1, 1)

        def pair(p, carry):
            c0 = 2 * p
            sfire(0)
            sdrain(0)

            @pl.when(c0 + 2 < n)
            def _():
                load(c0 + 2, 0)

            sfire(1)

            @pl.when(c0 + 2 < n)
            def _():
                gfire(c0 + 2, 0)

            sdrain(1)

            @pl.when(c0 + 3 < n)
            def _():
                load(c0 + 3, 1)
                gfire(c0 + 3, 1)

            return carry

        lax.fori_loop(0, n // 2, pair, 0)

        @pl.when(n % 2 == 1)
        def _():
            sfire(0)
            sdrain(0)

        plsc.subcore_barrier()

        @pl.when(cid == 0)
        def _():
            _node_copy(sid, lambda b2, n2: acc.at[pl.ds(b2, n2)],
                       lambda b2, n2: out0_hbm.at[pl.ds(b2, n2)])

        @pl.when(cid == 1)
        def _():
            _node_copy(sid, lambda b2, n2: acc.at[pl.ds(b2, n2)],
                       lambda b2, n2: out1_hbm.at[pl.ds(b2, n2)])

    return _agg


_agg1_kernel = _make_agg_kernel(True)
_agg2_kernel = _make_agg_kernel(False)


def _gelu(x):
    return 0.5 * x * (1.0 + lax.erf(x * 0.7071067811865476))


def _ln(x, g, b, eps=1e-5):
    m = jnp.mean(x, axis=-1, keepdims=True)
    v = jnp.mean((x - m) ** 2, axis=-1, keepdims=True)
    return (x - m) / jnp.sqrt(v + eps) * g + b


_RN = 1984
_RP = _RN // 8
_GRID = (_N + _RN - 1) // _RN


def _pack16(z, scale8=None):
    z3 = z.reshape(_RP, 8, _O)
    parts = []
    for s in range(8):
        p = z3[:, s, :]
        if scale8 is not None:
            p = p * scale8[:, s:s + 1]
        parts.append(p)
    return jnp.concatenate(parts, axis=1)


def _unpack16(zp):
    parts = [zp[:, 16 * s:16 * (s + 1)].reshape(_RP, 1, _O) for s in range(8)]
    return jnp.concatenate(parts, axis=1).reshape(_RN, _O)


def _full(shape):
    return pl.BlockSpec(shape, lambda i: tuple(0 for _ in shape))


def _enc_body(x0, x1, d0, d1, W10, b10, g10, bb10, W11, b11, g11, bb11,
              Wb0, b2t0, rg0, rb0, Wb1, b2t1, rg1, rb1, dsel, kJ,
              zcp, dinv8):
    di8 = lax.rsqrt(d0[...] + d1[...] + 1.0)
    dscale = jnp.dot(di8, dsel[...], preferred_element_type=jnp.float32)

    def hidden(x, W1, b1, g1, bb1):
        h = jnp.dot(x, W1, preferred_element_type=jnp.float32) + b1
        return _gelu(_ln(h, g1, bb1))

    h0 = hidden(x0[...], W10[...], b10[...], g10[...], bb10[...])
    h1 = hidden(x1[...], W11[...], b11[...], g11[...], bb11[...])
    h3 = jnp.concatenate([h0, h1], axis=1).reshape(_RP, 8, _D)
    hP = jnp.concatenate([h3[:, s, :] for s in range(8)], axis=1)

    def head(Wb, b2t, rg, rb):
        h2 = jnp.dot(hP, Wb[...], preferred_element_type=jnp.float32) + b2t[...]
        t = h2 + h2
        m = jnp.dot(t, kJ[...], preferred_element_type=jnp.float32)
        c = t - m
        v = jnp.dot(c * c, kJ[...], preferred_element_type=jnp.float32)
        return (c * lax.rsqrt(v + 1e-5) * rg[...] + rb[...]) * dscale

    zcp[0] = head(Wb0, b2t0, rg0, rb0)
    zcp[1] = head(Wb1, b2t1, rg1, rb1)
    dinv8[...] = di8


def _enc_call(x0, x1, d0, d1, w):
    grid = _GRID
    in_specs = [
        pl.BlockSpec((_RN, _D), lambda i: (i, 0)),
        pl.BlockSpec((_RN, _D), lambda i: (i, 0)),
        pl.BlockSpec((_RP, 8), lambda i: (i, 0)),
        pl.BlockSpec((_RP, 8), lambda i: (i, 0)),
    ] + [_full(a.shape) for a in w]
    return pl.pallas_call(
        _enc_body,
        grid=(grid,),
        in_specs=in_specs,
        out_specs=[pl.BlockSpec((_NC, _RP, 128), lambda i: (0, i, 0)),
                   pl.BlockSpec((_RP, 8), lambda i: (i, 0))],
        out_shape=[jax.ShapeDtypeStruct((_NC, _P, 128), jnp.float32),
                   jax.ShapeDtypeStruct((_P, 8), jnp.float32)],
    )(x0, x1, d0, d1, *w)


def _mid_body(S0p, S1p, dinv8, dsel, combk0, combk1, combb, kJ, g128, bb128,
              clfW1k, clfb1, clfW2k, clfb2, zp_pk, zpp_out, domp_out):
    dscale = jnp.dot(dinv8[...], dsel[...],
                     preferred_element_type=jnp.float32)
    t = (jnp.dot(S0p[...] * dscale, combk0[...],
                 preferred_element_type=jnp.float32)
         + jnp.dot(S1p[...] * dscale, combk1[...],
                   preferred_element_type=jnp.float32) + combb[...])
    m = jnp.dot(t, kJ[...], preferred_element_type=jnp.float32)
    c = t - m
    v = jnp.dot(c * c, kJ[...], preferred_element_type=jnp.float32)
    zP = c * lax.rsqrt(v + 1e-5) * g128[...] + bb128[...]
    h = _gelu(jnp.dot(zP, clfW1k[...], preferred_element_type=jnp.float32)
              + clfb1[...])
    domP = jnp.dot(h, clfW2k[...], preferred_element_type=jnp.float32) \
        + clfb2[...]
    zp_pk[...] = zP
    zpp_out[...] = zP * dscale
    domp_out[...] = domP


def _mid_call(S0p, S1p, dinv8, w):
    grid = _GRID
    in_specs = [
        pl.BlockSpec((_RP, 128), lambda i: (i, 0)),
        pl.BlockSpec((_RP, 128), lambda i: (i, 0)),
        pl.BlockSpec((_RP, 8), lambda i: (i, 0)),
    ] + [_full(a.shape) for a in w]
    return pl.pallas_call(
        _mid_body,
        grid=(grid,),
        in_specs=in_specs,
        out_specs=[pl.BlockSpec((_RP, 128), lambda i: (i, 0)),
                   pl.BlockSpec((_RP, 128), lambda i: (i, 0)),
                   pl.BlockSpec((_RP, 64), lambda i: (i, 0))],
        out_shape=[jax.ShapeDtypeStruct((_P, 128), jnp.float32),
                   jax.ShapeDtypeStruct((_P, 128), jnp.float32),
                   jax.ShapeDtypeStruct((_P, 64), jnp.float32)],
    )(S0p, S1p, dinv8, *w)


def _fin_body(S2a, S2b, zpp, dinv8, dsel, W0, b0, W1, b1, rec0, rec1):
    dscale = jnp.dot(dinv8[...], dsel[...],
                     preferred_element_type=jnp.float32)
    sum_p = (S2a[...] + S2b[...] - zpp[...]) * dscale
    aggz = _unpack16(sum_p)
    rec0[...] = jnp.dot(aggz, W0[...], preferred_element_type=jnp.float32) + b0[...]
    rec1[...] = jnp.dot(aggz, W1[...], preferred_element_type=jnp.float32) + b1[...]


def _fin_call(S2a, S2b, zpp, dinv8, w):
    grid = _GRID
    in_specs = [
        pl.BlockSpec((_RP, 128), lambda i: (i, 0)),
        pl.BlockSpec((_RP, 128), lambda i: (i, 0)),
        pl.BlockSpec((_RP, 128), lambda i: (i, 0)),
        pl.BlockSpec((_RP, 8), lambda i: (i, 0)),
    ] + [_full(a.shape) for a in w]
    return pl.pallas_call(
        _fin_body,
        grid=(grid,),
        in_specs=in_specs,
        out_specs=[pl.BlockSpec((_RN, _D), lambda i: (i, 0)),
                   pl.BlockSpec((_RN, _D), lambda i: (i, 0))],
        out_shape=[jax.ShapeDtypeStruct((_N, _D), jnp.float32),
                   jax.ShapeDtypeStruct((_N, _D), jnp.float32)],
    )(S2a, S2b, zpp, dinv8, *w)


def kernel(x0, x1, edge_index, enc0_W1, enc0_b1, enc0_g1, enc0_bb1, enc0_W2,
           enc0_b2, enc0_rg, enc0_rb, enc1_W1, enc1_b1, enc1_g1, enc1_bb1,
           enc1_W2, enc1_b2, enc1_rg, enc1_rb, comb_W, comb_b, comb_g,
           comb_bb, dec0_W, dec0_b, dec1_W, dec1_b, clf_W1, clf_b1, clf_W2,
           clf_b2):
    f32 = jnp.float32
    ei3 = edge_index.reshape(2, _ER, 128)
    zcol = jnp.zeros((_N,), f32)
    ones128 = jnp.ones((128,), f32)

    deg0, deg1 = _deg_kernel(ei3, zcol, ones128)

    row = lambda a: a.reshape(1, -1)
    eye8 = jnp.eye(8, dtype=f32)
    kron8 = lambda W: jnp.kron(eye8, W)
    tile8 = lambda v: jnp.tile(v, 8).reshape(1, -1)
    dsel = kron8(jnp.ones((1, _O), f32))
    kJ = kron8(jnp.full((_O, _O), 1.0 / _O, f32))
    z64 = jnp.zeros((_H, _O), f32)
    enc_w = (enc0_W1, row(enc0_b1), row(enc0_g1), row(enc0_bb1),
             enc1_W1, row(enc1_b1), row(enc1_g1), row(enc1_bb1),
             kron8(jnp.concatenate([enc0_W2, z64], axis=0)), tile8(enc0_b2),
             tile8(enc0_rg), tile8(enc0_rb),
             kron8(jnp.concatenate([z64, enc1_W2], axis=0)), tile8(enc1_b2),
             tile8(enc1_rg), tile8(enc1_rb), dsel, kJ)
    zcp_p, dinv8 = _enc_call(x0, x1, deg0.reshape(_P, 8), deg1.reshape(_P, 8),
                             enc_w)

    S0, S1 = _agg1_kernel(zcp_p.reshape(_NC, _N, _O), ei3)

    mid_w = (dsel, kron8(comb_W[:_O]), kron8(comb_W[_O:]), tile8(comb_b),
             kJ, tile8(comb_g), tile8(comb_bb), kron8(clf_W1), tile8(clf_b1),
             kron8(clf_W2), tile8(clf_b2))
    zP, zpP, domP = _mid_call(S0.reshape(_P, 128), S1.reshape(_P, 128),
                              dinv8, mid_w)

    S2a, S2b = _agg2_kernel(zpP.reshape(_N, _O), ei3)

    rec0, rec1 = _fin_call(S2a.reshape(_P, 128), S2b.reshape(_P, 128),
                           zpP, dinv8,
                           (dsel, dec0_W, row(dec0_b), dec1_W, row(dec1_b)))
    return (zP.reshape(_N, _O), rec0, rec1, domP.reshape(_N, 8))
